# Initial kernel scaffold; baseline (speedup 1.0000x reference)
import jax
import jax.numpy as jnp
from jax.experimental import pallas as pl


def kernel(x, edge_index, batch, emb, W1, b1, W2, b2, W3, b3, fcW1, fcb1, fcW2, fcb2, outW, outb):
    raise NotImplementedError("write your pallas kernel here")


# speedup vs baseline: 11.2703x; 11.2703x over previous
import functools

import jax
import jax.numpy as jnp
from jax import lax
from jax.experimental import pallas as pl
from jax.experimental.pallas import tpu as pltpu
from jax.experimental.pallas import tpu_sc as plsc

N = 9984
E = 319488
D = 128
G = 64
NC = 2
NS = 16
NW = NC * NS
EPT = E // NW
K = 128
NCH = EPT // K
RPT = N // NS
GPT = N // NW
KG = 104


def _emb_deg_body(idx_hbm, dst_hbm, emb_hbm,
                  h0_hbm, degp_hbm, idx_v, rows_v, dst_v, hist_v, red_v,
                  hist_sh, sem):
    c = lax.axis_index("c")
    s = lax.axis_index("s")
    wid = c * NS + s

    def zbody(i, carry):
        hist_v[pl.ds(i * 16, 16)] = jnp.zeros((16,), jnp.float32)
        return carry
    lax.fori_loop(0, N // 16, zbody, 0)

    def gbody(i, carry):
        off = wid * GPT + i * KG
        pltpu.sync_copy(idx_hbm.at[pl.ds(off, KG)], idx_v)
        pltpu.async_copy(emb_hbm.at[idx_v], rows_v, sem).wait()
        pltpu.sync_copy(rows_v, h0_hbm.at[pl.ds(off, KG)])
        return carry
    lax.fori_loop(0, GPT // KG, gbody, 0)

    ones = jnp.ones((16,), jnp.float32)
    def dbody(i, carry):
        off = wid * EPT + i * K
        pltpu.sync_copy(dst_hbm.at[pl.ds(off, K)], dst_v)
        for j in range(K // 16):
            plsc.addupdate_scatter(hist_v, [dst_v[pl.ds(j * 16, 16)]], ones)
        return carry
    lax.fori_loop(0, NCH, dbody, 0)

    pltpu.sync_copy(hist_v, hist_sh.at[pl.ds(s * N, N)])
    plsc.subcore_barrier()
    def zb2(i, carry):
        hist_v[pl.ds(s * RPT + i * 16, 16)] = jnp.zeros((16,), jnp.float32)
        return carry
    lax.fori_loop(0, RPT // 16, zb2, 0)
    def rbody(t, carry):
        pltpu.sync_copy(hist_sh.at[pl.ds(t * N + s * RPT, RPT)], red_v)
        def addb(i, carry2):
            sl = pl.ds(s * RPT + i * 16, 16)
            hist_v[sl] = hist_v[sl] + red_v[pl.ds(i * 16, 16)]
            return carry2
        lax.fori_loop(0, RPT // 16, addb, 0)
        return carry
    lax.fori_loop(0, NS, rbody, 0)
    pltpu.sync_copy(hist_v.at[pl.ds(s * RPT, RPT)],
                    degp_hbm.at[pl.ds(c * N + s * RPT, RPT)])


def _propagate_body(hs_hbm, src_hbm, dst_hbm, zeros_hbm, accp_hbm,
                    src_v, dst_v, rows_v, acc_sh, sem):
    c = lax.axis_index("c")
    s = lax.axis_index("s")
    wid = c * NS + s
    pltpu.sync_copy(zeros_hbm.at[pl.ds(s * RPT, RPT)],
                    acc_sh.at[pl.ds(s * RPT, RPT)])
    plsc.subcore_barrier()

    def body(i, carry):
        off = wid * EPT + i * K
        pltpu.sync_copy(src_hbm.at[pl.ds(off, K)], src_v)
        pltpu.sync_copy(dst_hbm.at[pl.ds(off, K)], dst_v)
        pltpu.async_copy(hs_hbm.at[src_v], rows_v, sem).wait()
        pltpu.sync_copy(rows_v, acc_sh.at[dst_v], add=True)
        return carry
    lax.fori_loop(0, NCH, body, 0)
    plsc.subcore_barrier()
    pltpu.sync_copy(acc_sh.at[pl.ds(s * RPT, RPT)],
                    accp_hbm.at[pl.ds(c * N + s * RPT, RPT)])


@functools.lru_cache(maxsize=None)
def _sc_kernels():
    mesh = plsc.VectorSubcoreMesh(core_axis_name="c", subcore_axis_name="s")
    emb_deg = pl.kernel(
        _emb_deg_body, mesh=mesh,
        out_type=[jax.ShapeDtypeStruct((N, D), jnp.float32),
                  jax.ShapeDtypeStruct((NC * N,), jnp.float32)],
        compiler_params=pltpu.CompilerParams(needs_layout_passes=False),
        scratch_types=[pltpu.VMEM((KG,), jnp.int32),
                       pltpu.VMEM((KG, D), jnp.float32),
                       pltpu.VMEM((K,), jnp.int32),
                       pltpu.VMEM((N,), jnp.float32),
                       pltpu.VMEM((RPT,), jnp.float32),
                       pltpu.VMEM_SHARED((NS * N,), jnp.float32),
                       pltpu.SemaphoreType.DMA])
    propagate = pl.kernel(
        _propagate_body, mesh=mesh,
        out_type=jax.ShapeDtypeStruct((NC * N, D), jnp.float32),
        scratch_types=[pltpu.VMEM((K,), jnp.int32),
                       pltpu.VMEM((K,), jnp.int32),
                       pltpu.VMEM((K, D), jnp.float32),
                       pltpu.VMEM_SHARED((N, D), jnp.float32),
                       pltpu.SemaphoreType.DMA])
    return emb_deg, propagate


def _sc_emb_deg(idx, dst, emb):
    return _sc_kernels()[0](idx, dst, emb)


def _sc_propagate(hs, src, dst, zeros):
    return _sc_kernels()[1](hs, src, dst, zeros)


_R = 1248
_RF = 768


def _dinv_block(dega, degb):
    deg = dega[:, :1] + degb[:, :1] + 1.0
    return lax.rsqrt(deg)


def _tc_first_body(h0_ref, w_ref, dega_ref, degb_ref, hw_ref, hs_ref):
    dinv = _dinv_block(dega_ref[...], degb_ref[...])
    hw = jnp.dot(h0_ref[...], w_ref[...], preferred_element_type=jnp.float32)
    hw_ref[...] = hw
    hs_ref[...] = dinv * hw


def _tc_mid_body(acca_ref, accb_ref, hw_ref, hprev_ref, b_ref,
                 dega_ref, degb_ref, w_ref,
                 h_ref, hwn_ref, hsn_ref):
    dinv = _dinv_block(dega_ref[...], degb_ref[...])
    hw = hw_ref[...]
    h = dinv * (acca_ref[...] + accb_ref[...]) + dinv * dinv * hw \
        + b_ref[...] + hprev_ref[...]
    h = jnp.maximum(h, 0.0)
    h_ref[...] = h
    hwn = jnp.dot(h, w_ref[...], preferred_element_type=jnp.float32)
    hwn_ref[...] = hwn
    hsn_ref[...] = dinv * hwn


def _tc_final_body(acca_ref, accb_ref, hw_ref, hprev_ref, b_ref,
                   dega_ref, degb_ref, batch_ref,
                   fw1_ref, fb1_ref, fw2_ref, fb2_ref, ow_ref, ob_ref,
                   out_ref, pool_scr, cnt_scr):
    pid = pl.program_id(0)

    @pl.when(pid == 0)
    def _init():
        pool_scr[...] = jnp.zeros((G, D), jnp.float32)
        cnt_scr[...] = jnp.zeros((G, D), jnp.float32)

    dinv = _dinv_block(dega_ref[...], degb_ref[...])
    hw = hw_ref[...]
    h3 = dinv * (acca_ref[...] + accb_ref[...]) + dinv * dinv * hw \
        + b_ref[...] + hprev_ref[...]

    gids = lax.broadcasted_iota(jnp.int32, (_RF, G), 1)
    ohb = (batch_ref[...] == gids).astype(jnp.float32)
    pool_scr[...] += lax.dot_general(
        ohb, h3, (((0,), (0,)), ((), ())),
        preferred_element_type=jnp.float32,
        precision=lax.Precision.HIGHEST)
    cnt_scr[...] += jnp.sum(ohb, axis=0)[:, None]

    @pl.when(pid == pl.num_programs(0) - 1)
    def _mlp():
        pooled = pool_scr[...] / jnp.maximum(cnt_scr[...], 1.0)
        r1 = jnp.maximum(jnp.dot(pooled, fw1_ref[...],
                                 preferred_element_type=jnp.float32)
                         + fb1_ref[...], 0.0)
        r2 = jnp.maximum(jnp.dot(r1, fw2_ref[...],
                                 preferred_element_type=jnp.float32)
                         + fb2_ref[...], 0.0)
        out_ref[...] = jnp.dot(r2, ow_ref[...],
                               preferred_element_type=jnp.float32) + ob_ref[...]


def _row_spec(r, cols):
    return pl.BlockSpec((r, cols), lambda i: (i, 0))


def _rep_spec(shape):
    nd = len(shape)
    return pl.BlockSpec(shape, lambda i: (0,) * nd)


def _tc_first(h0, W, dega, degb):
    grid = N // _R
    return pl.pallas_call(
        _tc_first_body,
        grid=(grid,),
        in_specs=[_row_spec(_R, D), _rep_spec((D, D)),
                  _row_spec(_R, 1), _row_spec(_R, 1)],
        out_specs=[_row_spec(_R, D), _row_spec(_R, D)],
        out_shape=[jax.ShapeDtypeStruct((N, D), jnp.float32),
                   jax.ShapeDtypeStruct((N, D), jnp.float32)],
    )(h0, W, dega, degb)


def _tc_mid(acca, accb, hw, hprev, b2d, dega, degb, Wn):
    grid = N // _R
    return pl.pallas_call(
        _tc_mid_body,
        grid=(grid,),
        in_specs=[_row_spec(_R, D), _row_spec(_R, D), _row_spec(_R, D),
                  _row_spec(_R, D), _rep_spec((1, D)),
                  _row_spec(_R, 1), _row_spec(_R, 1), _rep_spec((D, D))],
        out_specs=[_row_spec(_R, D), _row_spec(_R, D), _row_spec(_R, D)],
        out_shape=[jax.ShapeDtypeStruct((N, D), jnp.float32),
                   jax.ShapeDtypeStruct((N, D), jnp.float32),
                   jax.ShapeDtypeStruct((N, D), jnp.float32)],
    )(acca, accb, hw, hprev, b2d, dega, degb, Wn)


def _tc_final(acca, accb, hw, hprev, b2d, dega, degb, batch3,
              fcW1, fcb1, fcW2, fcb2, outWp, outb2):
    grid = N // _RF
    return pl.pallas_call(
        _tc_final_body,
        grid=(grid,),
        in_specs=[_row_spec(_RF, D), _row_spec(_RF, D), _row_spec(_RF, D),
                  _row_spec(_RF, D), _rep_spec((1, D)),
                  _row_spec(_RF, 1), _row_spec(_RF, 1),
                  _row_spec(_RF, 1),
                  _rep_spec((D, D)), _rep_spec((1, D)),
                  _rep_spec((D, G)), _rep_spec((1, G)),
                  _rep_spec((G, D)), _rep_spec((1, D))],
        out_specs=pl.BlockSpec((G, D), lambda i: (0, 0)),
        out_shape=jax.ShapeDtypeStruct((G, D), jnp.float32),
        scratch_shapes=[pltpu.VMEM((G, D), jnp.float32),
                        pltpu.VMEM((G, D), jnp.float32)],
    )(acca, accb, hw, hprev, b2d, dega, degb, batch3,
      fcW1, fcb1, fcW2, fcb2, outWp, outb2)


def kernel(x, edge_index, batch, emb, W1, b1, W2, b2, W3, b3,
           fcW1, fcb1, fcW2, fcb2, outW, outb):
    idx = jnp.nonzero(x, size=int(x.size), fill_value=0)[1].astype(jnp.int32)
    src = edge_index[0].astype(jnp.int32)
    dst = edge_index[1].astype(jnp.int32)

    zeros128 = jnp.zeros((N, D), jnp.float32)

    h0, degp = _sc_emb_deg(idx, dst, emb)
    dega = degp[:N].reshape(N, 1)
    degb = degp[N:].reshape(N, 1)

    b1r = b1.reshape(1, D)
    b2r = b2.reshape(1, D)
    b3r = b3.reshape(1, D)
    batch3 = batch.astype(jnp.int32).reshape(N, 1)
    outWp = jnp.pad(outW, ((0, 0), (0, D - outW.shape[1])))
    outb2 = jnp.pad(outb.reshape(1, 1), ((0, 0), (0, D - 1)))

    hw1, hs1 = _tc_first(h0, W1, dega, degb)

    accp1 = _sc_propagate(hs1, src, dst, zeros128)
    h1, hw2, hs2 = _tc_mid(accp1[:N], accp1[N:], hw1, h0, b1r, dega, degb, W2)

    accp2 = _sc_propagate(hs2, src, dst, zeros128)
    h2, hw3, hs3 = _tc_mid(accp2[:N], accp2[N:], hw2, h1, b2r, dega, degb, W3)

    accp3 = _sc_propagate(hs3, src, dst, zeros128)
    outp = _tc_final(accp3[:N], accp3[N:], hw3, h2, b3r, dega, degb, batch3,
                     fcW1, fcb1.reshape(1, D), fcW2,
                     jnp.pad(fcb2.reshape(1, G), ((0, 0), (0, 0))), outWp, outb2)
    return outp[:, :1]


# speedup vs baseline: 15.7211x; 1.3949x over previous
import functools

import jax
import jax.numpy as jnp
from jax import lax
from jax.experimental import pallas as pl
from jax.experimental.pallas import tpu as pltpu
from jax.experimental.pallas import tpu_sc as plsc

N = 9984
E = 319488
D = 128
G = 64
NC = 2
NS = 16
NW = NC * NS
EPT = E // NW
K = 128
NCH = EPT // K
RPT = N // NS
GPT = N // NW
KG = 104


def _emb_deg_body(idx_hbm, dst_hbm, emb_hbm,
                  h0_hbm, degp_hbm, idx_v, rows_v, dst_v, hist_v, red_v,
                  hist_sh, sem):
    c = lax.axis_index("c")
    s = lax.axis_index("s")
    wid = c * NS + s

    def zbody(i, carry):
        hist_v[pl.ds(i * 16, 16)] = jnp.zeros((16,), jnp.float32)
        return carry
    lax.fori_loop(0, N // 16, zbody, 0)

    def gbody(i, carry):
        off = wid * GPT + i * KG
        pltpu.sync_copy(idx_hbm.at[pl.ds(off, KG)], idx_v)
        pltpu.async_copy(emb_hbm.at[idx_v], rows_v, sem).wait()
        pltpu.sync_copy(rows_v, h0_hbm.at[pl.ds(off, KG)])
        return carry
    lax.fori_loop(0, GPT // KG, gbody, 0)

    ones = jnp.ones((16,), jnp.float32)
    def dbody(i, carry):
        off = wid * EPT + i * K
        pltpu.sync_copy(dst_hbm.at[pl.ds(off, K)], dst_v)
        for j in range(K // 16):
            plsc.addupdate_scatter(hist_v, [dst_v[pl.ds(j * 16, 16)]], ones)
        return carry
    lax.fori_loop(0, NCH, dbody, 0)

    pltpu.sync_copy(hist_v, hist_sh.at[pl.ds(s * N, N)])
    plsc.subcore_barrier()
    def zb2(i, carry):
        hist_v[pl.ds(s * RPT + i * 16, 16)] = jnp.zeros((16,), jnp.float32)
        return carry
    lax.fori_loop(0, RPT // 16, zb2, 0)
    def rbody(t, carry):
        pltpu.sync_copy(hist_sh.at[pl.ds(t * N + s * RPT, RPT)], red_v)
        def addb(i, carry2):
            sl = pl.ds(s * RPT + i * 16, 16)
            hist_v[sl] = hist_v[sl] + red_v[pl.ds(i * 16, 16)]
            return carry2
        lax.fori_loop(0, RPT // 16, addb, 0)
        return carry
    lax.fori_loop(0, NS, rbody, 0)
    pltpu.sync_copy(hist_v.at[pl.ds(s * RPT, RPT)],
                    degp_hbm.at[pl.ds(c * N + s * RPT, RPT)])


_NB = 3


def _propagate_body(hs_hbm, src_hbm, dst_hbm, zeros_hbm, accp_hbm,
                    sv0, sv1, sv2, dv0, dv1, dv2, r0, r1, r2,
                    sa0, sa1, sa2, sb0, sb1, sb2, sg0, sg1, sg2, acc_sh):
    src_v = [sv0, sv1, sv2]
    dst_v = [dv0, dv1, dv2]
    rows = [r0, r1, r2]
    sema = [sa0, sa1, sa2]
    semb = [sb0, sb1, sb2]
    semg = [sg0, sg1, sg2]
    c = lax.axis_index("c")
    s = lax.axis_index("s")
    wid = c * NS + s
    pltpu.sync_copy(zeros_hbm.at[pl.ds(s * RPT, RPT)],
                    acc_sh.at[pl.ds(s * RPT, RPT)])
    plsc.subcore_barrier()

    def group(gi, carry):
        base = wid * EPT + gi * (_NB * K)
        ha, hb, hg = [], [], []
        for b in range(_NB):
            off = base + b * K
            ha.append(pltpu.async_copy(src_hbm.at[pl.ds(off, K)],
                                       src_v[b], sema[b]))
            hb.append(pltpu.async_copy(dst_hbm.at[pl.ds(off, K)],
                                       dst_v[b], semb[b]))
        for b in range(_NB):
            ha[b].wait()
            hg.append(pltpu.async_copy(hs_hbm.at[src_v[b]], rows[b], semg[b]))
        for b in range(_NB):
            hg[b].wait()
            hb[b].wait()
            pltpu.sync_copy(rows[b], acc_sh.at[dst_v[b]], add=True)
        return carry
    lax.fori_loop(0, NCH // _NB, group, 0)
    plsc.subcore_barrier()
    pltpu.sync_copy(acc_sh.at[pl.ds(s * RPT, RPT)],
                    accp_hbm.at[pl.ds(c * N + s * RPT, RPT)])


@functools.lru_cache(maxsize=None)
def _sc_kernels():
    mesh = plsc.VectorSubcoreMesh(core_axis_name="c", subcore_axis_name="s")
    emb_deg = pl.kernel(
        _emb_deg_body, mesh=mesh,
        out_type=[jax.ShapeDtypeStruct((N, D), jnp.float32),
                  jax.ShapeDtypeStruct((NC * N,), jnp.float32)],
        compiler_params=pltpu.CompilerParams(needs_layout_passes=False),
        scratch_types=[pltpu.VMEM((KG,), jnp.int32),
                       pltpu.VMEM((KG, D), jnp.float32),
                       pltpu.VMEM((K,), jnp.int32),
                       pltpu.VMEM((N,), jnp.float32),
                       pltpu.VMEM((RPT,), jnp.float32),
                       pltpu.VMEM_SHARED((NS * N,), jnp.float32),
                       pltpu.SemaphoreType.DMA])
    propagate = pl.kernel(
        _propagate_body, mesh=mesh,
        out_type=jax.ShapeDtypeStruct((NC * N, D), jnp.float32),
        scratch_types=(
            [pltpu.VMEM((K,), jnp.int32)] * (2 * _NB)
            + [pltpu.VMEM((K, D), jnp.float32)] * _NB
            + [pltpu.SemaphoreType.DMA] * (3 * _NB)
            + [pltpu.VMEM_SHARED((N, D), jnp.float32)]))
    return emb_deg, propagate


def _sc_emb_deg(idx, dst, emb):
    return _sc_kernels()[0](idx, dst, emb)


def _sc_propagate(hs, src, dst, zeros):
    return _sc_kernels()[1](hs, src, dst, zeros)


_R = 1248
_RF = 768


def _dinv_block(dega, degb):
    deg = dega[:, :1] + degb[:, :1] + 1.0
    return lax.rsqrt(deg)


def _tc_first_body(h0_ref, w_ref, dega_ref, degb_ref, hw_ref, hs_ref):
    dinv = _dinv_block(dega_ref[...], degb_ref[...])
    hw = jnp.dot(h0_ref[...], w_ref[...], preferred_element_type=jnp.float32)
    hw_ref[...] = hw
    hs_ref[...] = dinv * hw


def _tc_mid_body(acca_ref, accb_ref, hw_ref, hprev_ref, b_ref,
                 dega_ref, degb_ref, w_ref,
                 h_ref, hwn_ref, hsn_ref):
    dinv = _dinv_block(dega_ref[...], degb_ref[...])
    hw = hw_ref[...]
    h = dinv * (acca_ref[...] + accb_ref[...]) + dinv * dinv * hw \
        + b_ref[...] + hprev_ref[...]
    h = jnp.maximum(h, 0.0)
    h_ref[...] = h
    hwn = jnp.dot(h, w_ref[...], preferred_element_type=jnp.float32)
    hwn_ref[...] = hwn
    hsn_ref[...] = dinv * hwn


def _tc_final_body(acca_ref, accb_ref, hw_ref, hprev_ref, b_ref,
                   dega_ref, degb_ref, batch_ref,
                   fw1_ref, fb1_ref, fw2_ref, fb2_ref, ow_ref, ob_ref,
                   out_ref, pool_scr, cnt_scr):
    pid = pl.program_id(0)

    @pl.when(pid == 0)
    def _init():
        pool_scr[...] = jnp.zeros((G, D), jnp.float32)
        cnt_scr[...] = jnp.zeros((G, D), jnp.float32)

    dinv = _dinv_block(dega_ref[...], degb_ref[...])
    hw = hw_ref[...]
    h3 = dinv * (acca_ref[...] + accb_ref[...]) + dinv * dinv * hw \
        + b_ref[...] + hprev_ref[...]

    gids = lax.broadcasted_iota(jnp.int32, (_RF, G), 1)
    ohb = (batch_ref[...] == gids).astype(jnp.float32)
    pool_scr[...] += lax.dot_general(
        ohb, h3, (((0,), (0,)), ((), ())),
        preferred_element_type=jnp.float32,
        precision=lax.Precision.HIGHEST)
    cnt_scr[...] += jnp.sum(ohb, axis=0)[:, None]

    @pl.when(pid == pl.num_programs(0) - 1)
    def _mlp():
        pooled = pool_scr[...] / jnp.maximum(cnt_scr[...], 1.0)
        r1 = jnp.maximum(jnp.dot(pooled, fw1_ref[...],
                                 preferred_element_type=jnp.float32)
                         + fb1_ref[...], 0.0)
        r2 = jnp.maximum(jnp.dot(r1, fw2_ref[...],
                                 preferred_element_type=jnp.float32)
                         + fb2_ref[...], 0.0)
        out_ref[...] = jnp.dot(r2, ow_ref[...],
                               preferred_element_type=jnp.float32) + ob_ref[...]


def _row_spec(r, cols):
    return pl.BlockSpec((r, cols), lambda i: (i, 0))


def _rep_spec(shape):
    nd = len(shape)
    return pl.BlockSpec(shape, lambda i: (0,) * nd)


def _tc_first(h0, W, dega, degb):
    grid = N // _R
    return pl.pallas_call(
        _tc_first_body,
        grid=(grid,),
        in_specs=[_row_spec(_R, D), _rep_spec((D, D)),
                  _row_spec(_R, 1), _row_spec(_R, 1)],
        out_specs=[_row_spec(_R, D), _row_spec(_R, D)],
        out_shape=[jax.ShapeDtypeStruct((N, D), jnp.float32),
                   jax.ShapeDtypeStruct((N, D), jnp.float32)],
    )(h0, W, dega, degb)


def _tc_mid(acca, accb, hw, hprev, b2d, dega, degb, Wn):
    grid = N // _R
    return pl.pallas_call(
        _tc_mid_body,
        grid=(grid,),
        in_specs=[_row_spec(_R, D), _row_spec(_R, D), _row_spec(_R, D),
                  _row_spec(_R, D), _rep_spec((1, D)),
                  _row_spec(_R, 1), _row_spec(_R, 1), _rep_spec((D, D))],
        out_specs=[_row_spec(_R, D), _row_spec(_R, D), _row_spec(_R, D)],
        out_shape=[jax.ShapeDtypeStruct((N, D), jnp.float32),
                   jax.ShapeDtypeStruct((N, D), jnp.float32),
                   jax.ShapeDtypeStruct((N, D), jnp.float32)],
    )(acca, accb, hw, hprev, b2d, dega, degb, Wn)


def _tc_final(acca, accb, hw, hprev, b2d, dega, degb, batch3,
              fcW1, fcb1, fcW2, fcb2, outWp, outb2):
    grid = N // _RF
    return pl.pallas_call(
        _tc_final_body,
        grid=(grid,),
        in_specs=[_row_spec(_RF, D), _row_spec(_RF, D), _row_spec(_RF, D),
                  _row_spec(_RF, D), _rep_spec((1, D)),
                  _row_spec(_RF, 1), _row_spec(_RF, 1),
                  _row_spec(_RF, 1),
                  _rep_spec((D, D)), _rep_spec((1, D)),
                  _rep_spec((D, G)), _rep_spec((1, G)),
                  _rep_spec((G, D)), _rep_spec((1, D))],
        out_specs=pl.BlockSpec((G, D), lambda i: (0, 0)),
        out_shape=jax.ShapeDtypeStruct((G, D), jnp.float32),
        scratch_shapes=[pltpu.VMEM((G, D), jnp.float32),
                        pltpu.VMEM((G, D), jnp.float32)],
    )(acca, accb, hw, hprev, b2d, dega, degb, batch3,
      fcW1, fcb1, fcW2, fcb2, outWp, outb2)


def kernel(x, edge_index, batch, emb, W1, b1, W2, b2, W3, b3,
           fcW1, fcb1, fcW2, fcb2, outW, outb):
    idx = jnp.nonzero(x, size=int(x.size), fill_value=0)[1].astype(jnp.int32)
    src = edge_index[0].astype(jnp.int32)
    dst = edge_index[1].astype(jnp.int32)

    zeros128 = jnp.zeros((N, D), jnp.float32)

    h0, degp = _sc_emb_deg(idx, dst, emb)
    dega = degp[:N].reshape(N, 1)
    degb = degp[N:].reshape(N, 1)

    b1r = b1.reshape(1, D)
    b2r = b2.reshape(1, D)
    b3r = b3.reshape(1, D)
    batch3 = batch.astype(jnp.int32).reshape(N, 1)
    outWp = jnp.pad(outW, ((0, 0), (0, D - outW.shape[1])))
    outb2 = jnp.pad(outb.reshape(1, 1), ((0, 0), (0, D - 1)))

    hw1, hs1 = _tc_first(h0, W1, dega, degb)

    accp1 = _sc_propagate(hs1, src, dst, zeros128)
    h1, hw2, hs2 = _tc_mid(accp1[:N], accp1[N:], hw1, h0, b1r, dega, degb, W2)

    accp2 = _sc_propagate(hs2, src, dst, zeros128)
    h2, hw3, hs3 = _tc_mid(accp2[:N], accp2[N:], hw2, h1, b2r, dega, degb, W3)

    accp3 = _sc_propagate(hs3, src, dst, zeros128)
    outp = _tc_final(accp3[:N], accp3[N:], hw3, h2, b3r, dega, degb, batch3,
                     fcW1, fcb1.reshape(1, D), fcW2,
                     jnp.pad(fcb2.reshape(1, G), ((0, 0), (0, 0))), outWp, outb2)
    return outp[:, :1]


# speedup vs baseline: 16.7483x; 1.0653x over previous
import functools

import jax
import jax.numpy as jnp
from jax import lax
from jax.experimental import pallas as pl
from jax.experimental.pallas import tpu as pltpu
from jax.experimental.pallas import tpu_sc as plsc

N = 9984
E = 319488
D = 128
G = 64
NC = 2
NS = 16
NW = NC * NS
EPT = E // NW
K = 128
NCH = EPT // K
RPT = N // NS
GPT = N // NW
KG = 104


def _emb_deg_body(idx_hbm, dst_hbm, emb_hbm,
                  h0_hbm, degp_hbm, idx_v, rows_v, dst_v, hist_v, red_v,
                  hist_sh, sem):
    c = lax.axis_index("c")
    s = lax.axis_index("s")
    wid = c * NS + s

    def zbody(i, carry):
        hist_v[pl.ds(i * 16, 16)] = jnp.zeros((16,), jnp.float32)
        return carry
    lax.fori_loop(0, N // 16, zbody, 0)

    def gbody(i, carry):
        off = wid * GPT + i * KG
        pltpu.sync_copy(idx_hbm.at[pl.ds(off, KG)], idx_v)
        pltpu.async_copy(emb_hbm.at[idx_v], rows_v, sem).wait()
        pltpu.sync_copy(rows_v, h0_hbm.at[pl.ds(off, KG)])
        return carry
    lax.fori_loop(0, GPT // KG, gbody, 0)

    ones = jnp.ones((16,), jnp.float32)
    def dbody(i, carry):
        off = wid * EPT + i * K
        pltpu.sync_copy(dst_hbm.at[pl.ds(off, K)], dst_v)
        for j in range(K // 16):
            plsc.addupdate_scatter(hist_v, [dst_v[pl.ds(j * 16, 16)]], ones)
        return carry
    lax.fori_loop(0, NCH, dbody, 0)

    pltpu.sync_copy(hist_v, hist_sh.at[pl.ds(s * N, N)])
    plsc.subcore_barrier()
    def zb2(i, carry):
        hist_v[pl.ds(s * RPT + i * 16, 16)] = jnp.zeros((16,), jnp.float32)
        return carry
    lax.fori_loop(0, RPT // 16, zb2, 0)
    def rbody(t, carry):
        pltpu.sync_copy(hist_sh.at[pl.ds(t * N + s * RPT, RPT)], red_v)
        def addb(i, carry2):
            sl = pl.ds(s * RPT + i * 16, 16)
            hist_v[sl] = hist_v[sl] + red_v[pl.ds(i * 16, 16)]
            return carry2
        lax.fori_loop(0, RPT // 16, addb, 0)
        return carry
    lax.fori_loop(0, NS, rbody, 0)
    pltpu.sync_copy(hist_v.at[pl.ds(s * RPT, RPT)],
                    degp_hbm.at[pl.ds(c * N + s * RPT, RPT)])


_NB = 3
_NG = 6


def _propagate_body(hs_hbm, src_hbm, dst_hbm, zeros_hbm, accp_hbm, *scr):
    src_v = list(scr[0:_NG])
    dst_v = list(scr[_NG:2 * _NG])
    rows = list(scr[2 * _NG:2 * _NG + _NB])
    o = 2 * _NG + _NB
    sema = list(scr[o:o + _NG])
    semb = list(scr[o + _NG:o + 2 * _NG])
    semg = list(scr[o + 2 * _NG:o + 2 * _NG + _NB])
    semsc = list(scr[o + 2 * _NG + _NB:o + 2 * _NG + 2 * _NB])
    acc_sh = scr[-1]
    c = lax.axis_index("c")
    s = lax.axis_index("s")
    wid = c * NS + s
    pltpu.sync_copy(zeros_hbm.at[pl.ds(s * RPT, RPT)],
                    acc_sh.at[pl.ds(s * RPT, RPT)])
    plsc.subcore_barrier()

    def group(gi, carry):
        base = wid * EPT + gi * (_NG * K)
        ha = [None] * _NG
        hb = [None] * _NG
        hg = [None] * _NG
        hs_ = [None] * _NG
        for j in range(_NB):
            off = base + j * K
            ha[j] = pltpu.async_copy(src_hbm.at[pl.ds(off, K)],
                                     src_v[j], sema[j])
            hb[j] = pltpu.async_copy(dst_hbm.at[pl.ds(off, K)],
                                     dst_v[j], semb[j])
        for j in range(_NG):
            rb = j % _NB
            if j + _NB < _NG:
                off = base + (j + _NB) * K
                ha[j + _NB] = pltpu.async_copy(src_hbm.at[pl.ds(off, K)],
                                               src_v[j + _NB], sema[j + _NB])
                hb[j + _NB] = pltpu.async_copy(dst_hbm.at[pl.ds(off, K)],
                                               dst_v[j + _NB], semb[j + _NB])
            ha[j].wait()
            if j >= _NB:
                hs_[j - _NB].wait()
            hg[j] = pltpu.async_copy(hs_hbm.at[src_v[j]], rows[rb], semg[rb])
            hg[j].wait()
            hb[j].wait()
            hs_[j] = pltpu.async_copy(rows[rb], acc_sh.at[dst_v[j]],
                                      semsc[rb], add=True)
        for j in range(_NG - _NB, _NG):
            hs_[j].wait()
        return carry
    lax.fori_loop(0, NCH // _NG, group, 0)
    plsc.subcore_barrier()
    pltpu.sync_copy(acc_sh.at[pl.ds(s * RPT, RPT)],
                    accp_hbm.at[pl.ds(c * N + s * RPT, RPT)])


@functools.lru_cache(maxsize=None)
def _sc_kernels():
    mesh = plsc.VectorSubcoreMesh(core_axis_name="c", subcore_axis_name="s")
    emb_deg = pl.kernel(
        _emb_deg_body, mesh=mesh,
        out_type=[jax.ShapeDtypeStruct((N, D), jnp.float32),
                  jax.ShapeDtypeStruct((NC * N,), jnp.float32)],
        compiler_params=pltpu.CompilerParams(needs_layout_passes=False),
        scratch_types=[pltpu.VMEM((KG,), jnp.int32),
                       pltpu.VMEM((KG, D), jnp.float32),
                       pltpu.VMEM((K,), jnp.int32),
                       pltpu.VMEM((N,), jnp.float32),
                       pltpu.VMEM((RPT,), jnp.float32),
                       pltpu.VMEM_SHARED((NS * N,), jnp.float32),
                       pltpu.SemaphoreType.DMA])
    propagate = pl.kernel(
        _propagate_body, mesh=mesh,
        out_type=jax.ShapeDtypeStruct((NC * N, D), jnp.float32),
        scratch_types=(
            [pltpu.VMEM((K,), jnp.int32)] * (2 * _NG)
            + [pltpu.VMEM((K, D), jnp.float32)] * _NB
            + [pltpu.SemaphoreType.DMA] * (2 * _NG + 2 * _NB)
            + [pltpu.VMEM_SHARED((N, D), jnp.float32)]))
    return emb_deg, propagate


def _sc_emb_deg(idx, dst, emb):
    return _sc_kernels()[0](idx, dst, emb)


def _sc_propagate(hs, src, dst, zeros):
    return _sc_kernels()[1](hs, src, dst, zeros)


_R = 1248
_RF = 768


def _dinv_block(dega, degb):
    deg = dega[:, :1] + degb[:, :1] + 1.0
    return lax.rsqrt(deg)


def _tc_first_body(h0_ref, w_ref, dega_ref, degb_ref, hw_ref, hs_ref):
    dinv = _dinv_block(dega_ref[...], degb_ref[...])
    hw = jnp.dot(h0_ref[...], w_ref[...], preferred_element_type=jnp.float32)
    hw_ref[...] = hw
    hs_ref[...] = dinv * hw


def _tc_mid_body(acca_ref, accb_ref, hw_ref, hprev_ref, b_ref,
                 dega_ref, degb_ref, w_ref,
                 h_ref, hwn_ref, hsn_ref):
    dinv = _dinv_block(dega_ref[...], degb_ref[...])
    hw = hw_ref[...]
    h = dinv * (acca_ref[...] + accb_ref[...]) + dinv * dinv * hw \
        + b_ref[...] + hprev_ref[...]
    h = jnp.maximum(h, 0.0)
    h_ref[...] = h
    hwn = jnp.dot(h, w_ref[...], preferred_element_type=jnp.float32)
    hwn_ref[...] = hwn
    hsn_ref[...] = dinv * hwn


def _tc_final_body(acca_ref, accb_ref, hw_ref, hprev_ref, b_ref,
                   dega_ref, degb_ref, batch_ref,
                   fw1_ref, fb1_ref, fw2_ref, fb2_ref, ow_ref, ob_ref,
                   out_ref, pool_scr, cnt_scr):
    pid = pl.program_id(0)

    @pl.when(pid == 0)
    def _init():
        pool_scr[...] = jnp.zeros((G, D), jnp.float32)
        cnt_scr[...] = jnp.zeros((G, D), jnp.float32)

    dinv = _dinv_block(dega_ref[...], degb_ref[...])
    hw = hw_ref[...]
    h3 = dinv * (acca_ref[...] + accb_ref[...]) + dinv * dinv * hw \
        + b_ref[...] + hprev_ref[...]

    gids = lax.broadcasted_iota(jnp.int32, (_RF, G), 1)
    ohb = (batch_ref[...] == gids).astype(jnp.float32)
    pool_scr[...] += lax.dot_general(
        ohb, h3, (((0,), (0,)), ((), ())),
        preferred_element_type=jnp.float32,
        precision=lax.Precision.HIGHEST)
    cnt_scr[...] += jnp.sum(ohb, axis=0)[:, None]

    @pl.when(pid == pl.num_programs(0) - 1)
    def _mlp():
        pooled = pool_scr[...] / jnp.maximum(cnt_scr[...], 1.0)
        r1 = jnp.maximum(jnp.dot(pooled, fw1_ref[...],
                                 preferred_element_type=jnp.float32)
                         + fb1_ref[...], 0.0)
        r2 = jnp.maximum(jnp.dot(r1, fw2_ref[...],
                                 preferred_element_type=jnp.float32)
                         + fb2_ref[...], 0.0)
        out_ref[...] = jnp.dot(r2, ow_ref[...],
                               preferred_element_type=jnp.float32) + ob_ref[...]


def _row_spec(r, cols):
    return pl.BlockSpec((r, cols), lambda i: (i, 0))


def _rep_spec(shape):
    nd = len(shape)
    return pl.BlockSpec(shape, lambda i: (0,) * nd)


def _tc_first(h0, W, dega, degb):
    grid = N // _R
    return pl.pallas_call(
        _tc_first_body,
        grid=(grid,),
        in_specs=[_row_spec(_R, D), _rep_spec((D, D)),
                  _row_spec(_R, 1), _row_spec(_R, 1)],
        out_specs=[_row_spec(_R, D), _row_spec(_R, D)],
        out_shape=[jax.ShapeDtypeStruct((N, D), jnp.float32),
                   jax.ShapeDtypeStruct((N, D), jnp.float32)],
    )(h0, W, dega, degb)


def _tc_mid(acca, accb, hw, hprev, b2d, dega, degb, Wn):
    grid = N // _R
    return pl.pallas_call(
        _tc_mid_body,
        grid=(grid,),
        in_specs=[_row_spec(_R, D), _row_spec(_R, D), _row_spec(_R, D),
                  _row_spec(_R, D), _rep_spec((1, D)),
                  _row_spec(_R, 1), _row_spec(_R, 1), _rep_spec((D, D))],
        out_specs=[_row_spec(_R, D), _row_spec(_R, D), _row_spec(_R, D)],
        out_shape=[jax.ShapeDtypeStruct((N, D), jnp.float32),
                   jax.ShapeDtypeStruct((N, D), jnp.float32),
                   jax.ShapeDtypeStruct((N, D), jnp.float32)],
    )(acca, accb, hw, hprev, b2d, dega, degb, Wn)


def _tc_final(acca, accb, hw, hprev, b2d, dega, degb, batch3,
              fcW1, fcb1, fcW2, fcb2, outWp, outb2):
    grid = N // _RF
    return pl.pallas_call(
        _tc_final_body,
        grid=(grid,),
        in_specs=[_row_spec(_RF, D), _row_spec(_RF, D), _row_spec(_RF, D),
                  _row_spec(_RF, D), _rep_spec((1, D)),
                  _row_spec(_RF, 1), _row_spec(_RF, 1),
                  _row_spec(_RF, 1),
                  _rep_spec((D, D)), _rep_spec((1, D)),
                  _rep_spec((D, G)), _rep_spec((1, G)),
                  _rep_spec((G, D)), _rep_spec((1, D))],
        out_specs=pl.BlockSpec((G, D), lambda i: (0, 0)),
        out_shape=jax.ShapeDtypeStruct((G, D), jnp.float32),
        scratch_shapes=[pltpu.VMEM((G, D), jnp.float32),
                        pltpu.VMEM((G, D), jnp.float32)],
    )(acca, accb, hw, hprev, b2d, dega, degb, batch3,
      fcW1, fcb1, fcW2, fcb2, outWp, outb2)


def kernel(x, edge_index, batch, emb, W1, b1, W2, b2, W3, b3,
           fcW1, fcb1, fcW2, fcb2, outW, outb):
    idx = jnp.nonzero(x, size=int(x.size), fill_value=0)[1].astype(jnp.int32)
    src = edge_index[0].astype(jnp.int32)
    dst = edge_index[1].astype(jnp.int32)

    zeros128 = jnp.zeros((N, D), jnp.float32)

    h0, degp = _sc_emb_deg(idx, dst, emb)
    dega = degp[:N].reshape(N, 1)
    degb = degp[N:].reshape(N, 1)

    b1r = b1.reshape(1, D)
    b2r = b2.reshape(1, D)
    b3r = b3.reshape(1, D)
    batch3 = batch.astype(jnp.int32).reshape(N, 1)
    outWp = jnp.pad(outW, ((0, 0), (0, D - outW.shape[1])))
    outb2 = jnp.pad(outb.reshape(1, 1), ((0, 0), (0, D - 1)))

    hw1, hs1 = _tc_first(h0, W1, dega, degb)

    accp1 = _sc_propagate(hs1, src, dst, zeros128)
    h1, hw2, hs2 = _tc_mid(accp1[:N], accp1[N:], hw1, h0, b1r, dega, degb, W2)

    accp2 = _sc_propagate(hs2, src, dst, zeros128)
    h2, hw3, hs3 = _tc_mid(accp2[:N], accp2[N:], hw2, h1, b2r, dega, degb, W3)

    accp3 = _sc_propagate(hs3, src, dst, zeros128)
    outp = _tc_final(accp3[:N], accp3[N:], hw3, h2, b3r, dega, degb, batch3,
                     fcW1, fcb1.reshape(1, D), fcW2,
                     jnp.pad(fcb2.reshape(1, G), ((0, 0), (0, 0))), outWp, outb2)
    return outp[:, :1]


# speedup vs baseline: 19.2860x; 1.1515x over previous
import functools

import jax
import jax.numpy as jnp
from jax import lax
from jax.experimental import pallas as pl
from jax.experimental.pallas import tpu as pltpu
from jax.experimental.pallas import tpu_sc as plsc

N = 9984
E = 319488
D = 128
G = 64
NC = 2
NS = 16
NW = NC * NS
EPT = E // NW
K = 128
NCH = EPT // K
RPT = N // NS
GPT = N // NW
KG = 104


def _emb_deg_body(idx_hbm, dst_hbm, emb_hbm,
                  h0_hbm, degp_hbm, idx_v, rows_v, dst_v, hist_v, red_v,
                  hist_sh, sem):
    c = lax.axis_index("c")
    s = lax.axis_index("s")
    wid = c * NS + s

    def zbody(i, carry):
        hist_v[pl.ds(i * 16, 16)] = jnp.zeros((16,), jnp.float32)
        return carry
    lax.fori_loop(0, N // 16, zbody, 0)

    def gbody(i, carry):
        off = wid * GPT + i * KG
        pltpu.sync_copy(idx_hbm.at[pl.ds(off, KG)], idx_v)
        pltpu.async_copy(emb_hbm.at[idx_v], rows_v, sem).wait()
        pltpu.sync_copy(rows_v, h0_hbm.at[pl.ds(off, KG)])
        return carry
    lax.fori_loop(0, GPT // KG, gbody, 0)

    ones = jnp.ones((16,), jnp.float32)
    def dbody(i, carry):
        off = wid * EPT + i * K
        pltpu.sync_copy(dst_hbm.at[pl.ds(off, K)], dst_v)
        for j in range(K // 16):
            plsc.addupdate_scatter(hist_v, [dst_v[pl.ds(j * 16, 16)]], ones)
        return carry
    lax.fori_loop(0, NCH, dbody, 0)

    pltpu.sync_copy(hist_v, hist_sh.at[pl.ds(s * N, N)])
    plsc.subcore_barrier()
    def zb2(i, carry):
        hist_v[pl.ds(s * RPT + i * 16, 16)] = jnp.zeros((16,), jnp.float32)
        return carry
    lax.fori_loop(0, RPT // 16, zb2, 0)
    def rbody(t, carry):
        pltpu.sync_copy(hist_sh.at[pl.ds(t * N + s * RPT, RPT)], red_v)
        def addb(i, carry2):
            sl = pl.ds(s * RPT + i * 16, 16)
            hist_v[sl] = hist_v[sl] + red_v[pl.ds(i * 16, 16)]
            return carry2
        lax.fori_loop(0, RPT // 16, addb, 0)
        return carry
    lax.fori_loop(0, NS, rbody, 0)
    pltpu.sync_copy(hist_v.at[pl.ds(s * RPT, RPT)],
                    degp_hbm.at[pl.ds(c * N + s * RPT, RPT)])


_NB = 3
_NG = 13
_NI = 6


def _propagate_body(hs_hbm, src_hbm, dst_hbm, zeros_hbm, accp_hbm, *scr):
    src_v = list(scr[0:_NI])
    dst_v = list(scr[_NI:2 * _NI])
    rows = list(scr[2 * _NI:2 * _NI + _NB])
    o = 2 * _NI + _NB
    sema = list(scr[o:o + _NI])
    semb = list(scr[o + _NI:o + 2 * _NI])
    semg = list(scr[o + 2 * _NI:o + 2 * _NI + _NB])
    semsc = list(scr[o + 2 * _NI + _NB:o + 2 * _NI + 2 * _NB])
    acc_sh = scr[-1]
    c = lax.axis_index("c")
    s = lax.axis_index("s")
    wid = c * NS + s
    pltpu.sync_copy(zeros_hbm.at[pl.ds(s * RPT, RPT)],
                    acc_sh.at[pl.ds(s * RPT, RPT)])
    plsc.subcore_barrier()

    def group(gi, carry):
        base = wid * EPT + gi * (_NG * K)
        ha = [None] * _NG
        hb = [None] * _NG
        hg = [None] * _NG
        hsc = [None] * _NG

        def sct(j):
            hg[j].wait()
            hb[j].wait()
            hsc[j] = pltpu.async_copy(rows[j % _NB],
                                      acc_sh.at[dst_v[j % _NI]],
                                      semsc[j % _NB], add=True)
        for j in range(_NB):
            off = base + j * K
            ha[j] = pltpu.async_copy(src_hbm.at[pl.ds(off, K)],
                                     src_v[j % _NI], sema[j % _NI])
            hb[j] = pltpu.async_copy(dst_hbm.at[pl.ds(off, K)],
                                     dst_v[j % _NI], semb[j % _NI])
        for j in range(_NG):
            rb = j % _NB
            if j >= _NB:
                hsc[j - _NB].wait()
            if j + _NB < _NG:
                off = base + (j + _NB) * K
                sl = (j + _NB) % _NI
                ha[j + _NB] = pltpu.async_copy(src_hbm.at[pl.ds(off, K)],
                                               src_v[sl], sema[sl])
                hb[j + _NB] = pltpu.async_copy(dst_hbm.at[pl.ds(off, K)],
                                               dst_v[sl], semb[sl])
            ha[j].wait()
            hg[j] = pltpu.async_copy(hs_hbm.at[src_v[j % _NI]],
                                     rows[rb], semg[rb])
            if j >= 1:
                sct(j - 1)
        sct(_NG - 1)
        for j in range(_NG - _NB, _NG):
            hsc[j].wait()
        return carry
    lax.fori_loop(0, NCH // _NG, group, 0)
    plsc.subcore_barrier()
    pltpu.sync_copy(acc_sh.at[pl.ds(s * RPT, RPT)],
                    accp_hbm.at[pl.ds(c * N + s * RPT, RPT)])


@functools.lru_cache(maxsize=None)
def _sc_kernels():
    mesh = plsc.VectorSubcoreMesh(core_axis_name="c", subcore_axis_name="s")
    emb_deg = pl.kernel(
        _emb_deg_body, mesh=mesh,
        out_type=[jax.ShapeDtypeStruct((N, D), jnp.float32),
                  jax.ShapeDtypeStruct((NC * N,), jnp.float32)],
        compiler_params=pltpu.CompilerParams(needs_layout_passes=False),
        scratch_types=[pltpu.VMEM((KG,), jnp.int32),
                       pltpu.VMEM((KG, D), jnp.float32),
                       pltpu.VMEM((K,), jnp.int32),
                       pltpu.VMEM((N,), jnp.float32),
                       pltpu.VMEM((RPT,), jnp.float32),
                       pltpu.VMEM_SHARED((NS * N,), jnp.float32),
                       pltpu.SemaphoreType.DMA])
    propagate = pl.kernel(
        _propagate_body, mesh=mesh,
        out_type=jax.ShapeDtypeStruct((NC * N, D), jnp.float32),
        scratch_types=(
            [pltpu.VMEM((K,), jnp.int32)] * (2 * _NI)
            + [pltpu.VMEM((K, D), jnp.float32)] * _NB
            + [pltpu.SemaphoreType.DMA] * (2 * _NI + 2 * _NB)
            + [pltpu.VMEM_SHARED((N, D), jnp.float32)]))
    return emb_deg, propagate


def _sc_emb_deg(idx, dst, emb):
    return _sc_kernels()[0](idx, dst, emb)


def _sc_propagate(hs, src, dst, zeros):
    return _sc_kernels()[1](hs, src, dst, zeros)


_R = 1248
_RF = 768


def _dinv_block(dega, degb):
    deg = dega[:, :1] + degb[:, :1] + 1.0
    return lax.rsqrt(deg)


def _tc_first_body(h0_ref, w_ref, dega_ref, degb_ref, hw_ref, hs_ref):
    dinv = _dinv_block(dega_ref[...], degb_ref[...])
    hw = jnp.dot(h0_ref[...], w_ref[...], preferred_element_type=jnp.float32)
    hw_ref[...] = hw
    hs_ref[...] = dinv * hw


def _tc_mid_body(acca_ref, accb_ref, hw_ref, hprev_ref, b_ref,
                 dega_ref, degb_ref, w_ref,
                 h_ref, hwn_ref, hsn_ref):
    dinv = _dinv_block(dega_ref[...], degb_ref[...])
    hw = hw_ref[...]
    h = dinv * (acca_ref[...] + accb_ref[...]) + dinv * dinv * hw \
        + b_ref[...] + hprev_ref[...]
    h = jnp.maximum(h, 0.0)
    h_ref[...] = h
    hwn = jnp.dot(h, w_ref[...], preferred_element_type=jnp.float32)
    hwn_ref[...] = hwn
    hsn_ref[...] = dinv * hwn


def _tc_final_body(acca_ref, accb_ref, hw_ref, hprev_ref, b_ref,
                   dega_ref, degb_ref, batch_ref,
                   fw1_ref, fb1_ref, fw2_ref, fb2_ref, ow_ref, ob_ref,
                   out_ref, pool_scr, cnt_scr):
    pid = pl.program_id(0)

    @pl.when(pid == 0)
    def _init():
        pool_scr[...] = jnp.zeros((G, D), jnp.float32)
        cnt_scr[...] = jnp.zeros((G, D), jnp.float32)

    dinv = _dinv_block(dega_ref[...], degb_ref[...])
    hw = hw_ref[...]
    h3 = dinv * (acca_ref[...] + accb_ref[...]) + dinv * dinv * hw \
        + b_ref[...] + hprev_ref[...]

    gids = lax.broadcasted_iota(jnp.int32, (_RF, G), 1)
    ohb = (batch_ref[...] == gids).astype(jnp.float32)
    pool_scr[...] += lax.dot_general(
        ohb, h3, (((0,), (0,)), ((), ())),
        preferred_element_type=jnp.float32,
        precision=lax.Precision.HIGHEST)
    cnt_scr[...] += jnp.sum(ohb, axis=0)[:, None]

    @pl.when(pid == pl.num_programs(0) - 1)
    def _mlp():
        pooled = pool_scr[...] / jnp.maximum(cnt_scr[...], 1.0)
        r1 = jnp.maximum(jnp.dot(pooled, fw1_ref[...],
                                 preferred_element_type=jnp.float32)
                         + fb1_ref[...], 0.0)
        r2 = jnp.maximum(jnp.dot(r1, fw2_ref[...],
                                 preferred_element_type=jnp.float32)
                         + fb2_ref[...], 0.0)
        out_ref[...] = jnp.dot(r2, ow_ref[...],
                               preferred_element_type=jnp.float32) + ob_ref[...]


def _row_spec(r, cols):
    return pl.BlockSpec((r, cols), lambda i: (i, 0))


def _rep_spec(shape):
    nd = len(shape)
    return pl.BlockSpec(shape, lambda i: (0,) * nd)


def _tc_first(h0, W, dega, degb):
    grid = N // _R
    return pl.pallas_call(
        _tc_first_body,
        grid=(grid,),
        in_specs=[_row_spec(_R, D), _rep_spec((D, D)),
                  _row_spec(_R, 1), _row_spec(_R, 1)],
        out_specs=[_row_spec(_R, D), _row_spec(_R, D)],
        out_shape=[jax.ShapeDtypeStruct((N, D), jnp.float32),
                   jax.ShapeDtypeStruct((N, D), jnp.float32)],
    )(h0, W, dega, degb)


def _tc_mid(acca, accb, hw, hprev, b2d, dega, degb, Wn):
    grid = N // _R
    return pl.pallas_call(
        _tc_mid_body,
        grid=(grid,),
        in_specs=[_row_spec(_R, D), _row_spec(_R, D), _row_spec(_R, D),
                  _row_spec(_R, D), _rep_spec((1, D)),
                  _row_spec(_R, 1), _row_spec(_R, 1), _rep_spec((D, D))],
        out_specs=[_row_spec(_R, D), _row_spec(_R, D), _row_spec(_R, D)],
        out_shape=[jax.ShapeDtypeStruct((N, D), jnp.float32),
                   jax.ShapeDtypeStruct((N, D), jnp.float32),
                   jax.ShapeDtypeStruct((N, D), jnp.float32)],
    )(acca, accb, hw, hprev, b2d, dega, degb, Wn)


def _tc_final(acca, accb, hw, hprev, b2d, dega, degb, batch3,
              fcW1, fcb1, fcW2, fcb2, outWp, outb2):
    grid = N // _RF
    return pl.pallas_call(
        _tc_final_body,
        grid=(grid,),
        in_specs=[_row_spec(_RF, D), _row_spec(_RF, D), _row_spec(_RF, D),
                  _row_spec(_RF, D), _rep_spec((1, D)),
                  _row_spec(_RF, 1), _row_spec(_RF, 1),
                  _row_spec(_RF, 1),
                  _rep_spec((D, D)), _rep_spec((1, D)),
                  _rep_spec((D, G)), _rep_spec((1, G)),
                  _rep_spec((G, D)), _rep_spec((1, D))],
        out_specs=pl.BlockSpec((G, D), lambda i: (0, 0)),
        out_shape=jax.ShapeDtypeStruct((G, D), jnp.float32),
        scratch_shapes=[pltpu.VMEM((G, D), jnp.float32),
                        pltpu.VMEM((G, D), jnp.float32)],
    )(acca, accb, hw, hprev, b2d, dega, degb, batch3,
      fcW1, fcb1, fcW2, fcb2, outWp, outb2)


def kernel(x, edge_index, batch, emb, W1, b1, W2, b2, W3, b3,
           fcW1, fcb1, fcW2, fcb2, outW, outb):
    idx = jnp.nonzero(x, size=int(x.size), fill_value=0)[1].astype(jnp.int32)
    src = edge_index[0].astype(jnp.int32)
    dst = edge_index[1].astype(jnp.int32)

    zeros128 = jnp.zeros((N, D), jnp.float32)

    h0, degp = _sc_emb_deg(idx, dst, emb)
    dega = degp[:N].reshape(N, 1)
    degb = degp[N:].reshape(N, 1)

    b1r = b1.reshape(1, D)
    b2r = b2.reshape(1, D)
    b3r = b3.reshape(1, D)
    batch3 = batch.astype(jnp.int32).reshape(N, 1)
    outWp = jnp.pad(outW, ((0, 0), (0, D - outW.shape[1])))
    outb2 = jnp.pad(outb.reshape(1, 1), ((0, 0), (0, D - 1)))

    hw1, hs1 = _tc_first(h0, W1, dega, degb)

    accp1 = _sc_propagate(hs1, src, dst, zeros128)
    h1, hw2, hs2 = _tc_mid(accp1[:N], accp1[N:], hw1, h0, b1r, dega, degb, W2)

    accp2 = _sc_propagate(hs2, src, dst, zeros128)
    h2, hw3, hs3 = _tc_mid(accp2[:N], accp2[N:], hw2, h1, b2r, dega, degb, W3)

    accp3 = _sc_propagate(hs3, src, dst, zeros128)
    outp = _tc_final(accp3[:N], accp3[N:], hw3, h2, b3r, dega, degb, batch3,
                     fcW1, fcb1.reshape(1, D), fcW2,
                     jnp.pad(fcb2.reshape(1, G), ((0, 0), (0, 0))), outWp, outb2)
    return outp[:, :1]


# speedup vs baseline: 20.5411x; 1.0651x over previous
import functools

import jax
import jax.numpy as jnp
from jax import lax
from jax.experimental import pallas as pl
from jax.experimental.pallas import tpu as pltpu
from jax.experimental.pallas import tpu_sc as plsc

N = 9984
E = 319488
D = 128
G = 64
NC = 2
NS = 16
NW = NC * NS
EPT = E // NW
K = 128
NCH = EPT // K
RPT = N // NS
GPT = N // NW
KG = 104


_ND = 6


def _emb_deg_body(idx_hbm, dst_hbm, emb_hbm, zeros1_hbm,
                  h0_hbm, degp_hbm, *scr):
    giv = list(scr[0:3])
    grv = list(scr[3:6])
    dvv = list(scr[6:6 + _ND])
    hist_v, rv0, rv1, acc_v = scr[6 + _ND:10 + _ND]
    hist_sh = scr[10 + _ND]
    sems = list(scr[11 + _ND:])
    sgv, sdv = sems[0:3], sems[3:3 + _ND]
    srv = sems[3 + _ND:6 + _ND]
    semz, semz2, sr0, sr1 = sems[6 + _ND:10 + _ND]
    semr = [sr0, sr1]
    rv = [rv0, rv1]
    c = lax.axis_index("c")
    s = lax.axis_index("s")
    wid = c * NS + s

    hz = pltpu.async_copy(zeros1_hbm.at[pl.ds(0, N)], hist_v, semz)
    hz2 = pltpu.async_copy(zeros1_hbm.at[pl.ds(0, RPT)], acc_v, semz2)

    hidx, hg = [], [None] * 3
    for i in range(3):
        off = wid * GPT + i * KG
        hidx.append(pltpu.async_copy(idx_hbm.at[pl.ds(off, KG)],
                                     giv[i], sgv[i]))
    for i in range(3):
        hidx[i].wait()
        hg[i] = pltpu.async_copy(emb_hbm.at[giv[i]], grv[i], sgv[i])
    hwb = [None] * 3
    for i in range(3):
        off = wid * GPT + i * KG
        hg[i].wait()
        hwb[i] = pltpu.async_copy(grv[i], h0_hbm.at[pl.ds(off, KG)], srv[i])

    hz.wait()
    ones = jnp.ones((16,), jnp.float32)

    def dgroup(g, carry):
        base = wid * EPT + g * (_ND * K)
        hd = []
        for u in range(_ND):
            hd.append(pltpu.async_copy(dst_hbm.at[pl.ds(base + u * K, K)],
                                       dvv[u], sdv[u]))
        for u in range(_ND):
            hd[u].wait()
            for j in range(K // 16):
                plsc.addupdate_scatter(hist_v,
                                       [dvv[u][pl.ds(j * 16, 16)]], ones)
        return carry
    lax.fori_loop(0, NCH // _ND, dgroup, 0)

    pltpu.sync_copy(hist_v, hist_sh.at[pl.ds(s * N, N)])
    plsc.subcore_barrier()
    hz2.wait()
    hr = [None] * (NS + 1)
    hr[0] = pltpu.async_copy(hist_sh.at[pl.ds(0 * N + s * RPT, RPT)],
                             rv[0], semr[0])
    for t in range(NS):
        hr[t].wait()
        if t + 1 < NS:
            hr[t + 1] = pltpu.async_copy(
                hist_sh.at[pl.ds((t + 1) * N + s * RPT, RPT)],
                rv[(t + 1) % 2], semr[(t + 1) % 2])
        buf = rv[t % 2]

        def addb(i, carry2):
            sl = pl.ds(i * 16, 16)
            acc_v[sl] = acc_v[sl] + buf[sl]
            return carry2
        lax.fori_loop(0, RPT // 16, addb, 0)
    for i in range(3):
        hwb[i].wait()
    pltpu.sync_copy(acc_v, degp_hbm.at[pl.ds(c * N + s * RPT, RPT)])


_NB = 3
_NG = 13
_NI = 6


def _propagate_body(hs_hbm, src_hbm, dst_hbm, zeros_hbm, accp_hbm, *scr):
    src_v = list(scr[0:_NI])
    dst_v = list(scr[_NI:2 * _NI])
    rows = list(scr[2 * _NI:2 * _NI + _NB])
    o = 2 * _NI + _NB
    sema = list(scr[o:o + _NI])
    semb = list(scr[o + _NI:o + 2 * _NI])
    semg = list(scr[o + 2 * _NI:o + 2 * _NI + _NB])
    semsc = list(scr[o + 2 * _NI + _NB:o + 2 * _NI + 2 * _NB])
    acc_sh = scr[-1]
    c = lax.axis_index("c")
    s = lax.axis_index("s")
    wid = c * NS + s
    pltpu.sync_copy(zeros_hbm.at[pl.ds(s * RPT, RPT)],
                    acc_sh.at[pl.ds(s * RPT, RPT)])
    plsc.subcore_barrier()

    def group(gi, carry):
        base = wid * EPT + gi * (_NG * K)
        ha = [None] * _NG
        hb = [None] * _NG
        hg = [None] * _NG
        hsc = [None] * _NG

        def sct(j):
            hg[j].wait()
            hb[j].wait()
            hsc[j] = pltpu.async_copy(rows[j % _NB],
                                      acc_sh.at[dst_v[j % _NI]],
                                      semsc[j % _NB], add=True)
        for j in range(_NB):
            off = base + j * K
            ha[j] = pltpu.async_copy(src_hbm.at[pl.ds(off, K)],
                                     src_v[j % _NI], sema[j % _NI])
            hb[j] = pltpu.async_copy(dst_hbm.at[pl.ds(off, K)],
                                     dst_v[j % _NI], semb[j % _NI])
        for j in range(_NG):
            rb = j % _NB
            if j >= _NB:
                hsc[j - _NB].wait()
            if j + _NB < _NG:
                off = base + (j + _NB) * K
                sl = (j + _NB) % _NI
                ha[j + _NB] = pltpu.async_copy(src_hbm.at[pl.ds(off, K)],
                                               src_v[sl], sema[sl])
                hb[j + _NB] = pltpu.async_copy(dst_hbm.at[pl.ds(off, K)],
                                               dst_v[sl], semb[sl])
            ha[j].wait()
            hg[j] = pltpu.async_copy(hs_hbm.at[src_v[j % _NI]],
                                     rows[rb], semg[rb])
            if j >= 1:
                sct(j - 1)
        sct(_NG - 1)
        for j in range(_NG - _NB, _NG):
            hsc[j].wait()
        return carry
    lax.fori_loop(0, NCH // _NG, group, 0)
    plsc.subcore_barrier()
    pltpu.sync_copy(acc_sh.at[pl.ds(s * RPT, RPT)],
                    accp_hbm.at[pl.ds(c * N + s * RPT, RPT)])


@functools.lru_cache(maxsize=None)
def _sc_kernels():
    mesh = plsc.VectorSubcoreMesh(core_axis_name="c", subcore_axis_name="s")
    emb_deg = pl.kernel(
        _emb_deg_body, mesh=mesh,
        out_type=[jax.ShapeDtypeStruct((N, D), jnp.float32),
                  jax.ShapeDtypeStruct((NC * N,), jnp.float32)],
        compiler_params=pltpu.CompilerParams(needs_layout_passes=False),
        scratch_types=(
            [pltpu.VMEM((KG,), jnp.int32)] * 3
            + [pltpu.VMEM((KG, D), jnp.float32)] * 3
            + [pltpu.VMEM((K,), jnp.int32)] * _ND
            + [pltpu.VMEM((N,), jnp.float32)]
            + [pltpu.VMEM((RPT,), jnp.float32)] * 3
            + [pltpu.VMEM_SHARED((NS * N,), jnp.float32)]
            + [pltpu.SemaphoreType.DMA] * (10 + _ND)))
    propagate = pl.kernel(
        _propagate_body, mesh=mesh,
        out_type=jax.ShapeDtypeStruct((NC * N, D), jnp.float32),
        scratch_types=(
            [pltpu.VMEM((K,), jnp.int32)] * (2 * _NI)
            + [pltpu.VMEM((K, D), jnp.float32)] * _NB
            + [pltpu.SemaphoreType.DMA] * (2 * _NI + 2 * _NB)
            + [pltpu.VMEM_SHARED((N, D), jnp.float32)]))
    return emb_deg, propagate


def _sc_emb_deg(idx, dst, emb, zeros1):
    return _sc_kernels()[0](idx, dst, emb, zeros1)


def _sc_propagate(hs, src, dst, zeros):
    return _sc_kernels()[1](hs, src, dst, zeros)


_R = 1248
_RF = 768


def _dinv_block(dega, degb):
    deg = dega[:, :1] + degb[:, :1] + 1.0
    return lax.rsqrt(deg)


def _tc_first_body(h0_ref, w_ref, dega_ref, degb_ref, hw_ref, hs_ref):
    dinv = _dinv_block(dega_ref[...], degb_ref[...])
    hw = jnp.dot(h0_ref[...], w_ref[...], preferred_element_type=jnp.float32)
    hw_ref[...] = hw
    hs_ref[...] = dinv * hw


def _tc_mid_body(acca_ref, accb_ref, hw_ref, hprev_ref, b_ref,
                 dega_ref, degb_ref, w_ref,
                 h_ref, hwn_ref, hsn_ref):
    dinv = _dinv_block(dega_ref[...], degb_ref[...])
    hw = hw_ref[...]
    h = dinv * (acca_ref[...] + accb_ref[...]) + dinv * dinv * hw \
        + b_ref[...] + hprev_ref[...]
    h = jnp.maximum(h, 0.0)
    h_ref[...] = h
    hwn = jnp.dot(h, w_ref[...], preferred_element_type=jnp.float32)
    hwn_ref[...] = hwn
    hsn_ref[...] = dinv * hwn


def _tc_final_body(acca_ref, accb_ref, hw_ref, hprev_ref, b_ref,
                   dega_ref, degb_ref, batch_ref,
                   fw1_ref, fb1_ref, fw2_ref, fb2_ref, ow_ref, ob_ref,
                   out_ref, pool_scr, cnt_scr):
    pid = pl.program_id(0)

    @pl.when(pid == 0)
    def _init():
        pool_scr[...] = jnp.zeros((G, D), jnp.float32)
        cnt_scr[...] = jnp.zeros((G, D), jnp.float32)

    dinv = _dinv_block(dega_ref[...], degb_ref[...])
    hw = hw_ref[...]
    h3 = dinv * (acca_ref[...] + accb_ref[...]) + dinv * dinv * hw \
        + b_ref[...] + hprev_ref[...]

    gids = lax.broadcasted_iota(jnp.int32, (_RF, G), 1)
    ohb = (batch_ref[...] == gids).astype(jnp.float32)
    pool_scr[...] += lax.dot_general(
        ohb, h3, (((0,), (0,)), ((), ())),
        preferred_element_type=jnp.float32,
        precision=lax.Precision.HIGHEST)
    cnt_scr[...] += jnp.sum(ohb, axis=0)[:, None]

    @pl.when(pid == pl.num_programs(0) - 1)
    def _mlp():
        pooled = pool_scr[...] / jnp.maximum(cnt_scr[...], 1.0)
        r1 = jnp.maximum(jnp.dot(pooled, fw1_ref[...],
                                 preferred_element_type=jnp.float32)
                         + fb1_ref[...], 0.0)
        r2 = jnp.maximum(jnp.dot(r1, fw2_ref[...],
                                 preferred_element_type=jnp.float32)
                         + fb2_ref[...], 0.0)
        out_ref[...] = jnp.dot(r2, ow_ref[...],
                               preferred_element_type=jnp.float32) + ob_ref[...]


def _row_spec(r, cols):
    return pl.BlockSpec((r, cols), lambda i: (i, 0))


def _rep_spec(shape):
    nd = len(shape)
    return pl.BlockSpec(shape, lambda i: (0,) * nd)


def _tc_first(h0, W, dega, degb):
    grid = N // _R
    return pl.pallas_call(
        _tc_first_body,
        grid=(grid,),
        in_specs=[_row_spec(_R, D), _rep_spec((D, D)),
                  _row_spec(_R, 1), _row_spec(_R, 1)],
        out_specs=[_row_spec(_R, D), _row_spec(_R, D)],
        out_shape=[jax.ShapeDtypeStruct((N, D), jnp.float32),
                   jax.ShapeDtypeStruct((N, D), jnp.float32)],
    )(h0, W, dega, degb)


def _tc_mid(acca, accb, hw, hprev, b2d, dega, degb, Wn):
    grid = N // _R
    return pl.pallas_call(
        _tc_mid_body,
        grid=(grid,),
        in_specs=[_row_spec(_R, D), _row_spec(_R, D), _row_spec(_R, D),
                  _row_spec(_R, D), _rep_spec((1, D)),
                  _row_spec(_R, 1), _row_spec(_R, 1), _rep_spec((D, D))],
        out_specs=[_row_spec(_R, D), _row_spec(_R, D), _row_spec(_R, D)],
        out_shape=[jax.ShapeDtypeStruct((N, D), jnp.float32),
                   jax.ShapeDtypeStruct((N, D), jnp.float32),
                   jax.ShapeDtypeStruct((N, D), jnp.float32)],
    )(acca, accb, hw, hprev, b2d, dega, degb, Wn)


def _tc_final(acca, accb, hw, hprev, b2d, dega, degb, batch3,
              fcW1, fcb1, fcW2, fcb2, outWp, outb2):
    grid = N // _RF
    return pl.pallas_call(
        _tc_final_body,
        grid=(grid,),
        in_specs=[_row_spec(_RF, D), _row_spec(_RF, D), _row_spec(_RF, D),
                  _row_spec(_RF, D), _rep_spec((1, D)),
                  _row_spec(_RF, 1), _row_spec(_RF, 1),
                  _row_spec(_RF, 1),
                  _rep_spec((D, D)), _rep_spec((1, D)),
                  _rep_spec((D, G)), _rep_spec((1, G)),
                  _rep_spec((G, D)), _rep_spec((1, D))],
        out_specs=pl.BlockSpec((G, D), lambda i: (0, 0)),
        out_shape=jax.ShapeDtypeStruct((G, D), jnp.float32),
        scratch_shapes=[pltpu.VMEM((G, D), jnp.float32),
                        pltpu.VMEM((G, D), jnp.float32)],
    )(acca, accb, hw, hprev, b2d, dega, degb, batch3,
      fcW1, fcb1, fcW2, fcb2, outWp, outb2)


def kernel(x, edge_index, batch, emb, W1, b1, W2, b2, W3, b3,
           fcW1, fcb1, fcW2, fcb2, outW, outb):
    idx = jnp.nonzero(x, size=int(x.size), fill_value=0)[1].astype(jnp.int32)
    src = edge_index[0].astype(jnp.int32)
    dst = edge_index[1].astype(jnp.int32)

    zeros128 = jnp.zeros((N, D), jnp.float32)
    zeros1 = jnp.zeros((N,), jnp.float32)

    h0, degp = _sc_emb_deg(idx, dst, emb, zeros1)
    dega = degp[:N].reshape(N, 1)
    degb = degp[N:].reshape(N, 1)

    b1r = b1.reshape(1, D)
    b2r = b2.reshape(1, D)
    b3r = b3.reshape(1, D)
    batch3 = batch.astype(jnp.int32).reshape(N, 1)
    outWp = jnp.pad(outW, ((0, 0), (0, D - outW.shape[1])))
    outb2 = jnp.pad(outb.reshape(1, 1), ((0, 0), (0, D - 1)))

    hw1, hs1 = _tc_first(h0, W1, dega, degb)

    accp1 = _sc_propagate(hs1, src, dst, zeros128)
    h1, hw2, hs2 = _tc_mid(accp1[:N], accp1[N:], hw1, h0, b1r, dega, degb, W2)

    accp2 = _sc_propagate(hs2, src, dst, zeros128)
    h2, hw3, hs3 = _tc_mid(accp2[:N], accp2[N:], hw2, h1, b2r, dega, degb, W3)

    accp3 = _sc_propagate(hs3, src, dst, zeros128)
    outp = _tc_final(accp3[:N], accp3[N:], hw3, h2, b3r, dega, degb, batch3,
                     fcW1, fcb1.reshape(1, D), fcW2,
                     jnp.pad(fcb2.reshape(1, G), ((0, 0), (0, 0))), outWp, outb2)
    return outp[:, :1]


# speedup vs baseline: 21.7324x; 1.0580x over previous
import functools

import jax
import jax.numpy as jnp
from jax import lax
from jax.experimental import pallas as pl
from jax.experimental.pallas import tpu as pltpu
from jax.experimental.pallas import tpu_sc as plsc

N = 9984
E = 319488
D = 128
G = 64
NC = 2
NS = 16
NW = NC * NS
EPT = E // NW
K = 128
NCH = EPT // K
RPT = N // NS
GPT = N // NW
KG = 104


_ND = 6


def _deg_body(dst_hbm, zeros1_hbm, degp_hbm, *scr):
    dvv = list(scr[0:_ND])
    hist_v, rv0, rv1, acc_v = scr[_ND:4 + _ND]
    hist_sh = scr[4 + _ND]
    sems = list(scr[5 + _ND:])
    sdv = sems[0:_ND]
    semz, semz2, sr0, sr1 = sems[_ND:4 + _ND]
    semr = [sr0, sr1]
    rv = [rv0, rv1]
    c = lax.axis_index("c")
    s = lax.axis_index("s")
    wid = c * NS + s

    hz = pltpu.async_copy(zeros1_hbm.at[pl.ds(0, N)], hist_v, semz)
    hz2 = pltpu.async_copy(zeros1_hbm.at[pl.ds(0, RPT)], acc_v, semz2)

    hz.wait()
    ones = jnp.ones((16,), jnp.float32)

    def dgroup(g, carry):
        base = wid * EPT + g * (_ND * K)
        hd = []
        for u in range(_ND):
            hd.append(pltpu.async_copy(dst_hbm.at[pl.ds(base + u * K, K)],
                                       dvv[u], sdv[u]))
        for u in range(_ND):
            hd[u].wait()
            for j in range(K // 16):
                plsc.addupdate_scatter(hist_v,
                                       [dvv[u][pl.ds(j * 16, 16)]], ones)
        return carry
    lax.fori_loop(0, NCH // _ND, dgroup, 0)

    pltpu.sync_copy(hist_v, hist_sh.at[pl.ds(s * N, N)])
    plsc.subcore_barrier()
    hz2.wait()
    hr = [None] * (NS + 1)
    hr[0] = pltpu.async_copy(hist_sh.at[pl.ds(0 * N + s * RPT, RPT)],
                             rv[0], semr[0])
    for t in range(NS):
        hr[t].wait()
        if t + 1 < NS:
            hr[t + 1] = pltpu.async_copy(
                hist_sh.at[pl.ds((t + 1) * N + s * RPT, RPT)],
                rv[(t + 1) % 2], semr[(t + 1) % 2])
        buf = rv[t % 2]

        def addb(i, carry2):
            sl = pl.ds(i * 16, 16)
            acc_v[sl] = acc_v[sl] + buf[sl]
            return carry2
        lax.fori_loop(0, RPT // 16, addb, 0)
    pltpu.sync_copy(acc_v, degp_hbm.at[pl.ds(c * N + s * RPT, RPT)])


_NB = 3
_NG = 13
_NI = 6


def _propagate_body(hs_hbm, src_hbm, dst_hbm, zeros_hbm, accp_hbm, *scr):
    src_v = list(scr[0:_NI])
    dst_v = list(scr[_NI:2 * _NI])
    rows = list(scr[2 * _NI:2 * _NI + _NB])
    o = 2 * _NI + _NB
    sema = list(scr[o:o + _NI])
    semb = list(scr[o + _NI:o + 2 * _NI])
    semg = list(scr[o + 2 * _NI:o + 2 * _NI + _NB])
    semsc = list(scr[o + 2 * _NI + _NB:o + 2 * _NI + 2 * _NB])
    acc_sh = scr[-1]
    c = lax.axis_index("c")
    s = lax.axis_index("s")
    wid = c * NS + s
    pltpu.sync_copy(zeros_hbm.at[pl.ds(s * RPT, RPT)],
                    acc_sh.at[pl.ds(s * RPT, RPT)])
    plsc.subcore_barrier()

    def group(gi, carry):
        base = wid * EPT + gi * (_NG * K)
        ha = [None] * _NG
        hb = [None] * _NG
        hg = [None] * _NG
        hsc = [None] * _NG

        def sct(j):
            hg[j].wait()
            hb[j].wait()
            hsc[j] = pltpu.async_copy(rows[j % _NB],
                                      acc_sh.at[dst_v[j % _NI]],
                                      semsc[j % _NB], add=True)
        for j in range(_NB):
            off = base + j * K
            ha[j] = pltpu.async_copy(src_hbm.at[pl.ds(off, K)],
                                     src_v[j % _NI], sema[j % _NI])
            hb[j] = pltpu.async_copy(dst_hbm.at[pl.ds(off, K)],
                                     dst_v[j % _NI], semb[j % _NI])
        for j in range(_NG):
            rb = j % _NB
            if j >= _NB:
                hsc[j - _NB].wait()
            if j + _NB < _NG:
                off = base + (j + _NB) * K
                sl = (j + _NB) % _NI
                ha[j + _NB] = pltpu.async_copy(src_hbm.at[pl.ds(off, K)],
                                               src_v[sl], sema[sl])
                hb[j + _NB] = pltpu.async_copy(dst_hbm.at[pl.ds(off, K)],
                                               dst_v[sl], semb[sl])
            ha[j].wait()
            hg[j] = pltpu.async_copy(hs_hbm.at[src_v[j % _NI]],
                                     rows[rb], semg[rb])
            if j >= 1:
                sct(j - 1)
        sct(_NG - 1)
        for j in range(_NG - _NB, _NG):
            hsc[j].wait()
        return carry
    lax.fori_loop(0, NCH // _NG, group, 0)
    plsc.subcore_barrier()
    pltpu.sync_copy(acc_sh.at[pl.ds(s * RPT, RPT)],
                    accp_hbm.at[pl.ds(c * N + s * RPT, RPT)])


@functools.lru_cache(maxsize=None)
def _sc_kernels():
    mesh = plsc.VectorSubcoreMesh(core_axis_name="c", subcore_axis_name="s")
    deg = pl.kernel(
        _deg_body, mesh=mesh,
        out_type=jax.ShapeDtypeStruct((NC * N,), jnp.float32),
        compiler_params=pltpu.CompilerParams(needs_layout_passes=False),
        scratch_types=(
            [pltpu.VMEM((K,), jnp.int32)] * _ND
            + [pltpu.VMEM((N,), jnp.float32)]
            + [pltpu.VMEM((RPT,), jnp.float32)] * 3
            + [pltpu.VMEM_SHARED((NS * N,), jnp.float32)]
            + [pltpu.SemaphoreType.DMA] * (4 + _ND)))
    propagate = pl.kernel(
        _propagate_body, mesh=mesh,
        out_type=jax.ShapeDtypeStruct((NC * N, D), jnp.float32),
        scratch_types=(
            [pltpu.VMEM((K,), jnp.int32)] * (2 * _NI)
            + [pltpu.VMEM((K, D), jnp.float32)] * _NB
            + [pltpu.SemaphoreType.DMA] * (2 * _NI + 2 * _NB)
            + [pltpu.VMEM_SHARED((N, D), jnp.float32)]))
    return deg, propagate


def _sc_deg(dst, zeros1):
    return _sc_kernels()[0](dst, zeros1)


def _sc_propagate(hs, src, dst, zeros):
    return _sc_kernels()[1](hs, src, dst, zeros)


_R = 1248
_RF = 768


def _dinv_block(dega, degb):
    deg = dega[:, :1] + degb[:, :1] + 1.0
    return lax.rsqrt(deg)


def _tc_embed_body(idx_ref, embp_ref, w_ref, h0_ref, hw_ref):
    cids = lax.broadcasted_iota(jnp.int32, (_R, 16), 1)
    oh = (idx_ref[...] == cids).astype(jnp.float32)
    h0 = jnp.dot(oh, embp_ref[...], preferred_element_type=jnp.float32,
                 precision=lax.Precision.HIGHEST)
    h0_ref[...] = h0
    hw_ref[...] = jnp.dot(h0, w_ref[...], preferred_element_type=jnp.float32)


def _tc_scale_body(hw_ref, dega_ref, degb_ref, hs_ref):
    dinv = _dinv_block(dega_ref[...], degb_ref[...])
    hs_ref[...] = dinv * hw_ref[...]


def _tc_mid_body(acca_ref, accb_ref, hw_ref, hprev_ref, b_ref,
                 dega_ref, degb_ref, w_ref,
                 h_ref, hwn_ref, hsn_ref):
    dinv = _dinv_block(dega_ref[...], degb_ref[...])
    hw = hw_ref[...]
    h = dinv * (acca_ref[...] + accb_ref[...]) + dinv * dinv * hw \
        + b_ref[...] + hprev_ref[...]
    h = jnp.maximum(h, 0.0)
    h_ref[...] = h
    hwn = jnp.dot(h, w_ref[...], preferred_element_type=jnp.float32)
    hwn_ref[...] = hwn
    hsn_ref[...] = dinv * hwn


def _tc_final_body(acca_ref, accb_ref, hw_ref, hprev_ref, b_ref,
                   dega_ref, degb_ref, batch_ref,
                   fw1_ref, fb1_ref, fw2_ref, fb2_ref, ow_ref, ob_ref,
                   out_ref, pool_scr, cnt_scr):
    pid = pl.program_id(0)

    @pl.when(pid == 0)
    def _init():
        pool_scr[...] = jnp.zeros((G, D), jnp.float32)
        cnt_scr[...] = jnp.zeros((G, D), jnp.float32)

    dinv = _dinv_block(dega_ref[...], degb_ref[...])
    hw = hw_ref[...]
    h3 = dinv * (acca_ref[...] + accb_ref[...]) + dinv * dinv * hw \
        + b_ref[...] + hprev_ref[...]

    gids = lax.broadcasted_iota(jnp.int32, (_RF, G), 1)
    ohb = (batch_ref[...] == gids).astype(jnp.float32)
    pool_scr[...] += lax.dot_general(
        ohb, h3, (((0,), (0,)), ((), ())),
        preferred_element_type=jnp.float32,
        precision=lax.Precision.HIGHEST)
    cnt_scr[...] += jnp.sum(ohb, axis=0)[:, None]

    @pl.when(pid == pl.num_programs(0) - 1)
    def _mlp():
        pooled = pool_scr[...] / jnp.maximum(cnt_scr[...], 1.0)
        r1 = jnp.maximum(jnp.dot(pooled, fw1_ref[...],
                                 preferred_element_type=jnp.float32)
                         + fb1_ref[...], 0.0)
        r2 = jnp.maximum(jnp.dot(r1, fw2_ref[...],
                                 preferred_element_type=jnp.float32)
                         + fb2_ref[...], 0.0)
        out_ref[...] = jnp.dot(r2, ow_ref[...],
                               preferred_element_type=jnp.float32) + ob_ref[...]


def _row_spec(r, cols):
    return pl.BlockSpec((r, cols), lambda i: (i, 0))


def _rep_spec(shape):
    nd = len(shape)
    return pl.BlockSpec(shape, lambda i: (0,) * nd)


def _tc_embed(idxcol, embp, W):
    grid = N // _R
    return pl.pallas_call(
        _tc_embed_body,
        grid=(grid,),
        in_specs=[_row_spec(_R, 1), _rep_spec((16, D)), _rep_spec((D, D))],
        out_specs=[_row_spec(_R, D), _row_spec(_R, D)],
        out_shape=[jax.ShapeDtypeStruct((N, D), jnp.float32),
                   jax.ShapeDtypeStruct((N, D), jnp.float32)],
    )(idxcol, embp, W)


def _tc_scale(hw, dega, degb):
    grid = N // _R
    return pl.pallas_call(
        _tc_scale_body,
        grid=(grid,),
        in_specs=[_row_spec(_R, D), _row_spec(_R, 1), _row_spec(_R, 1)],
        out_specs=_row_spec(_R, D),
        out_shape=jax.ShapeDtypeStruct((N, D), jnp.float32),
    )(hw, dega, degb)


def _tc_mid(acca, accb, hw, hprev, b2d, dega, degb, Wn):
    grid = N // _R
    return pl.pallas_call(
        _tc_mid_body,
        grid=(grid,),
        in_specs=[_row_spec(_R, D), _row_spec(_R, D), _row_spec(_R, D),
                  _row_spec(_R, D), _rep_spec((1, D)),
                  _row_spec(_R, 1), _row_spec(_R, 1), _rep_spec((D, D))],
        out_specs=[_row_spec(_R, D), _row_spec(_R, D), _row_spec(_R, D)],
        out_shape=[jax.ShapeDtypeStruct((N, D), jnp.float32),
                   jax.ShapeDtypeStruct((N, D), jnp.float32),
                   jax.ShapeDtypeStruct((N, D), jnp.float32)],
    )(acca, accb, hw, hprev, b2d, dega, degb, Wn)


def _tc_final(acca, accb, hw, hprev, b2d, dega, degb, batch3,
              fcW1, fcb1, fcW2, fcb2, outWp, outb2):
    grid = N // _RF
    return pl.pallas_call(
        _tc_final_body,
        grid=(grid,),
        in_specs=[_row_spec(_RF, D), _row_spec(_RF, D), _row_spec(_RF, D),
                  _row_spec(_RF, D), _rep_spec((1, D)),
                  _row_spec(_RF, 1), _row_spec(_RF, 1),
                  _row_spec(_RF, 1),
                  _rep_spec((D, D)), _rep_spec((1, D)),
                  _rep_spec((D, G)), _rep_spec((1, G)),
                  _rep_spec((G, D)), _rep_spec((1, D))],
        out_specs=pl.BlockSpec((G, D), lambda i: (0, 0)),
        out_shape=jax.ShapeDtypeStruct((G, D), jnp.float32),
        scratch_shapes=[pltpu.VMEM((G, D), jnp.float32),
                        pltpu.VMEM((G, D), jnp.float32)],
    )(acca, accb, hw, hprev, b2d, dega, degb, batch3,
      fcW1, fcb1, fcW2, fcb2, outWp, outb2)


def kernel(x, edge_index, batch, emb, W1, b1, W2, b2, W3, b3,
           fcW1, fcb1, fcW2, fcb2, outW, outb):
    idx = jnp.nonzero(x, size=int(x.size), fill_value=0)[1].astype(jnp.int32)
    src = edge_index[0].astype(jnp.int32)
    dst = edge_index[1].astype(jnp.int32)

    zeros128 = jnp.zeros((N, D), jnp.float32)
    zeros1 = jnp.zeros((N,), jnp.float32)

    degp = _sc_deg(dst, zeros1)
    dega = degp[:N].reshape(N, 1)
    degb = degp[N:].reshape(N, 1)
    embp = jnp.pad(emb, ((0, 16 - emb.shape[0]), (0, 0)))
    h0, hw1 = _tc_embed(idx.reshape(N, 1), embp, W1)

    b1r = b1.reshape(1, D)
    b2r = b2.reshape(1, D)
    b3r = b3.reshape(1, D)
    batch3 = batch.astype(jnp.int32).reshape(N, 1)
    outWp = jnp.pad(outW, ((0, 0), (0, D - outW.shape[1])))
    outb2 = jnp.pad(outb.reshape(1, 1), ((0, 0), (0, D - 1)))

    hs1 = _tc_scale(hw1, dega, degb)

    accp1 = _sc_propagate(hs1, src, dst, zeros128)
    h1, hw2, hs2 = _tc_mid(accp1[:N], accp1[N:], hw1, h0, b1r, dega, degb, W2)

    accp2 = _sc_propagate(hs2, src, dst, zeros128)
    h2, hw3, hs3 = _tc_mid(accp2[:N], accp2[N:], hw2, h1, b2r, dega, degb, W3)

    accp3 = _sc_propagate(hs3, src, dst, zeros128)
    outp = _tc_final(accp3[:N], accp3[N:], hw3, h2, b3r, dega, degb, batch3,
                     fcW1, fcb1.reshape(1, D), fcW2,
                     jnp.pad(fcb2.reshape(1, G), ((0, 0), (0, 0))), outWp, outb2)
    return outp[:, :1]


# speedup vs baseline: 22.4723x; 1.0340x over previous
import functools

import jax
import jax.numpy as jnp
from jax import lax
from jax.experimental import pallas as pl
from jax.experimental.pallas import tpu as pltpu
from jax.experimental.pallas import tpu_sc as plsc

N = 9984
E = 319488
D = 128
G = 64
NC = 2
NS = 16
NW = NC * NS
EPT = E // NW
K = 128
NCH = EPT // K
RPT = N // NS
GPT = N // NW
KG = 104


_ND = 6


def _deg_body(dst_hbm, zeros1_hbm, degp_hbm, *scr):
    dvv = list(scr[0:_ND])
    hist_v, rv0, rv1, acc_v = scr[_ND:4 + _ND]
    hist_sh = scr[4 + _ND]
    sems = list(scr[5 + _ND:])
    sdv = sems[0:_ND]
    semz, semz2, sr0, sr1 = sems[_ND:4 + _ND]
    semr = [sr0, sr1]
    rv = [rv0, rv1]
    c = lax.axis_index("c")
    s = lax.axis_index("s")
    wid = c * NS + s

    hz = pltpu.async_copy(zeros1_hbm.at[pl.ds(0, N)], hist_v, semz)
    hz2 = pltpu.async_copy(zeros1_hbm.at[pl.ds(0, RPT)], acc_v, semz2)

    hz.wait()
    ones = jnp.ones((16,), jnp.float32)

    def dgroup(g, carry):
        base = wid * EPT + g * (_ND * K)
        hd = []
        for u in range(_ND):
            hd.append(pltpu.async_copy(dst_hbm.at[pl.ds(base + u * K, K)],
                                       dvv[u], sdv[u]))
        for u in range(_ND):
            hd[u].wait()
            for j in range(K // 16):
                plsc.addupdate_scatter(hist_v,
                                       [dvv[u][pl.ds(j * 16, 16)]], ones)
        return carry
    lax.fori_loop(0, NCH // _ND, dgroup, 0)

    pltpu.sync_copy(hist_v, hist_sh.at[pl.ds(s * N, N)])
    plsc.subcore_barrier()
    hz2.wait()
    hr = [None] * (NS + 1)
    hr[0] = pltpu.async_copy(hist_sh.at[pl.ds(0 * N + s * RPT, RPT)],
                             rv[0], semr[0])
    for t in range(NS):
        hr[t].wait()
        if t + 1 < NS:
            hr[t + 1] = pltpu.async_copy(
                hist_sh.at[pl.ds((t + 1) * N + s * RPT, RPT)],
                rv[(t + 1) % 2], semr[(t + 1) % 2])
        buf = rv[t % 2]

        def addb(i, carry2):
            sl = pl.ds(i * 16, 16)
            acc_v[sl] = acc_v[sl] + buf[sl]
            return carry2
        lax.fori_loop(0, RPT // 16, addb, 0)
    pltpu.sync_copy(acc_v, degp_hbm.at[pl.ds(c * N + s * RPT, RPT)])


_NB = 3
_NG = 26
_NI = 6


def _propagate_body(hs_hbm, src_hbm, dst_hbm, zeros_hbm, accp_hbm, *scr):
    src_v = list(scr[0:_NI])
    dst_v = list(scr[_NI:2 * _NI])
    rows = list(scr[2 * _NI:2 * _NI + _NB])
    o = 2 * _NI + _NB
    sema = list(scr[o:o + _NI])
    semb = list(scr[o + _NI:o + 2 * _NI])
    semg = list(scr[o + 2 * _NI:o + 2 * _NI + _NB])
    semsc = list(scr[o + 2 * _NI + _NB:o + 2 * _NI + 2 * _NB])
    acc_sh = scr[-1]
    c = lax.axis_index("c")
    s = lax.axis_index("s")
    wid = c * NS + s
    pltpu.sync_copy(zeros_hbm.at[pl.ds(s * RPT, RPT)],
                    acc_sh.at[pl.ds(s * RPT, RPT)])
    plsc.subcore_barrier()

    def group(gi, carry):
        base = wid * EPT + gi * (_NG * K)
        ha = [None] * _NG
        hb = [None] * _NG
        hg = [None] * _NG
        hsc = [None] * _NG

        def sct(j):
            hg[j].wait()
            hb[j].wait()
            hsc[j] = pltpu.async_copy(rows[j % _NB],
                                      acc_sh.at[dst_v[j % _NI]],
                                      semsc[j % _NB], add=True)
        for j in range(_NB):
            off = base + j * K
            ha[j] = pltpu.async_copy(src_hbm.at[pl.ds(off, K)],
                                     src_v[j % _NI], sema[j % _NI])
            hb[j] = pltpu.async_copy(dst_hbm.at[pl.ds(off, K)],
                                     dst_v[j % _NI], semb[j % _NI])
        for j in range(_NG):
            rb = j % _NB
            if j >= _NB:
                hsc[j - _NB].wait()
            if j + _NB < _NG:
                off = base + (j + _NB) * K
                sl = (j + _NB) % _NI
                ha[j + _NB] = pltpu.async_copy(src_hbm.at[pl.ds(off, K)],
                                               src_v[sl], sema[sl])
                hb[j + _NB] = pltpu.async_copy(dst_hbm.at[pl.ds(off, K)],
                                               dst_v[sl], semb[sl])
            ha[j].wait()
            hg[j] = pltpu.async_copy(hs_hbm.at[src_v[j % _NI]],
                                     rows[rb], semg[rb])
            if j >= 1:
                sct(j - 1)
        sct(_NG - 1)
        for j in range(_NG - _NB, _NG):
            hsc[j].wait()
        return carry
    lax.fori_loop(0, NCH // _NG, group, 0)
    plsc.subcore_barrier()
    pltpu.sync_copy(acc_sh.at[pl.ds(s * RPT, RPT)],
                    accp_hbm.at[pl.ds(c * N + s * RPT, RPT)])


@functools.lru_cache(maxsize=None)
def _sc_kernels():
    mesh = plsc.VectorSubcoreMesh(core_axis_name="c", subcore_axis_name="s")
    deg = pl.kernel(
        _deg_body, mesh=mesh,
        out_type=jax.ShapeDtypeStruct((NC * N,), jnp.float32),
        compiler_params=pltpu.CompilerParams(needs_layout_passes=False),
        scratch_types=(
            [pltpu.VMEM((K,), jnp.int32)] * _ND
            + [pltpu.VMEM((N,), jnp.float32)]
            + [pltpu.VMEM((RPT,), jnp.float32)] * 3
            + [pltpu.VMEM_SHARED((NS * N,), jnp.float32)]
            + [pltpu.SemaphoreType.DMA] * (4 + _ND)))
    propagate = pl.kernel(
        _propagate_body, mesh=mesh,
        out_type=jax.ShapeDtypeStruct((NC * N, D), jnp.float32),
        scratch_types=(
            [pltpu.VMEM((K,), jnp.int32)] * (2 * _NI)
            + [pltpu.VMEM((K, D), jnp.float32)] * _NB
            + [pltpu.SemaphoreType.DMA] * (2 * _NI + 2 * _NB)
            + [pltpu.VMEM_SHARED((N, D), jnp.float32)]))
    return deg, propagate


def _sc_deg(dst, zeros1):
    return _sc_kernels()[0](dst, zeros1)


def _sc_propagate(hs, src, dst, zeros):
    return _sc_kernels()[1](hs, src, dst, zeros)


_R = 1248
_RF = 768


def _dinv_block(dega, degb):
    deg = dega[:, :1] + degb[:, :1] + 1.0
    return lax.rsqrt(deg)


def _tc_embed_body(idx_ref, embp_ref, w_ref, h0_ref, hw_ref):
    cids = lax.broadcasted_iota(jnp.int32, (_R, 16), 1)
    oh = (idx_ref[...] == cids).astype(jnp.float32)
    h0 = jnp.dot(oh, embp_ref[...], preferred_element_type=jnp.float32,
                 precision=lax.Precision.HIGHEST)
    h0_ref[...] = h0
    hw_ref[...] = jnp.dot(h0, w_ref[...], preferred_element_type=jnp.float32)


def _tc_scale_body(hw_ref, dega_ref, degb_ref, hs_ref):
    dinv = _dinv_block(dega_ref[...], degb_ref[...])
    hs_ref[...] = dinv * hw_ref[...]


def _tc_mid_body(acca_ref, accb_ref, hw_ref, hprev_ref, b_ref,
                 dega_ref, degb_ref, w_ref,
                 h_ref, hwn_ref, hsn_ref):
    dinv = _dinv_block(dega_ref[...], degb_ref[...])
    hw = hw_ref[...]
    h = dinv * (acca_ref[...] + accb_ref[...]) + dinv * dinv * hw \
        + b_ref[...] + hprev_ref[...]
    h = jnp.maximum(h, 0.0)
    h_ref[...] = h
    hwn = jnp.dot(h, w_ref[...], preferred_element_type=jnp.float32)
    hwn_ref[...] = hwn
    hsn_ref[...] = dinv * hwn


def _tc_final_body(acca_ref, accb_ref, hw_ref, hprev_ref, b_ref,
                   dega_ref, degb_ref, batch_ref,
                   fw1_ref, fb1_ref, fw2_ref, fb2_ref, ow_ref, ob_ref,
                   out_ref, pool_scr, cnt_scr):
    pid = pl.program_id(0)

    @pl.when(pid == 0)
    def _init():
        pool_scr[...] = jnp.zeros((G, D), jnp.float32)
        cnt_scr[...] = jnp.zeros((G, D), jnp.float32)

    dinv = _dinv_block(dega_ref[...], degb_ref[...])
    hw = hw_ref[...]
    h3 = dinv * (acca_ref[...] + accb_ref[...]) + dinv * dinv * hw \
        + b_ref[...] + hprev_ref[...]

    gids = lax.broadcasted_iota(jnp.int32, (_RF, G), 1)
    ohb = (batch_ref[...] == gids).astype(jnp.float32)
    pool_scr[...] += lax.dot_general(
        ohb, h3, (((0,), (0,)), ((), ())),
        preferred_element_type=jnp.float32,
        precision=lax.Precision.HIGHEST)
    cnt_scr[...] += jnp.sum(ohb, axis=0)[:, None]

    @pl.when(pid == pl.num_programs(0) - 1)
    def _mlp():
        pooled = pool_scr[...] / jnp.maximum(cnt_scr[...], 1.0)
        r1 = jnp.maximum(jnp.dot(pooled, fw1_ref[...],
                                 preferred_element_type=jnp.float32)
                         + fb1_ref[...], 0.0)
        r2 = jnp.maximum(jnp.dot(r1, fw2_ref[...],
                                 preferred_element_type=jnp.float32)
                         + fb2_ref[...], 0.0)
        out_ref[...] = jnp.dot(r2, ow_ref[...],
                               preferred_element_type=jnp.float32) + ob_ref[...]


def _row_spec(r, cols):
    return pl.BlockSpec((r, cols), lambda i: (i, 0))


def _rep_spec(shape):
    nd = len(shape)
    return pl.BlockSpec(shape, lambda i: (0,) * nd)


def _tc_embed(idxcol, embp, W):
    grid = N // _R
    return pl.pallas_call(
        _tc_embed_body,
        grid=(grid,),
        in_specs=[_row_spec(_R, 1), _rep_spec((16, D)), _rep_spec((D, D))],
        out_specs=[_row_spec(_R, D), _row_spec(_R, D)],
        out_shape=[jax.ShapeDtypeStruct((N, D), jnp.float32),
                   jax.ShapeDtypeStruct((N, D), jnp.float32)],
    )(idxcol, embp, W)


def _tc_scale(hw, dega, degb):
    grid = N // _R
    return pl.pallas_call(
        _tc_scale_body,
        grid=(grid,),
        in_specs=[_row_spec(_R, D), _row_spec(_R, 1), _row_spec(_R, 1)],
        out_specs=_row_spec(_R, D),
        out_shape=jax.ShapeDtypeStruct((N, D), jnp.float32),
    )(hw, dega, degb)


def _tc_mid(acca, accb, hw, hprev, b2d, dega, degb, Wn):
    grid = N // _R
    return pl.pallas_call(
        _tc_mid_body,
        grid=(grid,),
        in_specs=[_row_spec(_R, D), _row_spec(_R, D), _row_spec(_R, D),
                  _row_spec(_R, D), _rep_spec((1, D)),
                  _row_spec(_R, 1), _row_spec(_R, 1), _rep_spec((D, D))],
        out_specs=[_row_spec(_R, D), _row_spec(_R, D), _row_spec(_R, D)],
        out_shape=[jax.ShapeDtypeStruct((N, D), jnp.float32),
                   jax.ShapeDtypeStruct((N, D), jnp.float32),
                   jax.ShapeDtypeStruct((N, D), jnp.float32)],
    )(acca, accb, hw, hprev, b2d, dega, degb, Wn)


def _tc_final(acca, accb, hw, hprev, b2d, dega, degb, batch3,
              fcW1, fcb1, fcW2, fcb2, outWp, outb2):
    grid = N // _RF
    return pl.pallas_call(
        _tc_final_body,
        grid=(grid,),
        in_specs=[_row_spec(_RF, D), _row_spec(_RF, D), _row_spec(_RF, D),
                  _row_spec(_RF, D), _rep_spec((1, D)),
                  _row_spec(_RF, 1), _row_spec(_RF, 1),
                  _row_spec(_RF, 1),
                  _rep_spec((D, D)), _rep_spec((1, D)),
                  _rep_spec((D, G)), _rep_spec((1, G)),
                  _rep_spec((G, D)), _rep_spec((1, D))],
        out_specs=pl.BlockSpec((G, D), lambda i: (0, 0)),
        out_shape=jax.ShapeDtypeStruct((G, D), jnp.float32),
        scratch_shapes=[pltpu.VMEM((G, D), jnp.float32),
                        pltpu.VMEM((G, D), jnp.float32)],
    )(acca, accb, hw, hprev, b2d, dega, degb, batch3,
      fcW1, fcb1, fcW2, fcb2, outWp, outb2)


def kernel(x, edge_index, batch, emb, W1, b1, W2, b2, W3, b3,
           fcW1, fcb1, fcW2, fcb2, outW, outb):
    idx = jnp.nonzero(x, size=int(x.size), fill_value=0)[1].astype(jnp.int32)
    src = edge_index[0].astype(jnp.int32)
    dst = edge_index[1].astype(jnp.int32)

    zeros128 = jnp.zeros((N, D), jnp.float32)
    zeros1 = jnp.zeros((N,), jnp.float32)

    degp = _sc_deg(dst, zeros1)
    dega = degp[:N].reshape(N, 1)
    degb = degp[N:].reshape(N, 1)
    embp = jnp.pad(emb, ((0, 16 - emb.shape[0]), (0, 0)))
    h0, hw1 = _tc_embed(idx.reshape(N, 1), embp, W1)

    b1r = b1.reshape(1, D)
    b2r = b2.reshape(1, D)
    b3r = b3.reshape(1, D)
    batch3 = batch.astype(jnp.int32).reshape(N, 1)
    outWp = jnp.pad(outW, ((0, 0), (0, D - outW.shape[1])))
    outb2 = jnp.pad(outb.reshape(1, 1), ((0, 0), (0, D - 1)))

    hs1 = _tc_scale(hw1, dega, degb)

    accp1 = _sc_propagate(hs1, src, dst, zeros128)
    h1, hw2, hs2 = _tc_mid(accp1[:N], accp1[N:], hw1, h0, b1r, dega, degb, W2)

    accp2 = _sc_propagate(hs2, src, dst, zeros128)
    h2, hw3, hs3 = _tc_mid(accp2[:N], accp2[N:], hw2, h1, b2r, dega, degb, W3)

    accp3 = _sc_propagate(hs3, src, dst, zeros128)
    outp = _tc_final(accp3[:N], accp3[N:], hw3, h2, b3r, dega, degb, batch3,
                     fcW1, fcb1.reshape(1, D), fcW2,
                     jnp.pad(fcb2.reshape(1, G), ((0, 0), (0, 0))), outWp, outb2)
    return outp[:, :1]


# speedup vs baseline: 22.9313x; 1.0204x over previous
import functools

import jax
import jax.numpy as jnp
from jax import lax
from jax.experimental import pallas as pl
from jax.experimental.pallas import tpu as pltpu
from jax.experimental.pallas import tpu_sc as plsc

N = 9984
E = 319488
D = 128
G = 64
NC = 2
NS = 16
NW = NC * NS
EPT = E // NW
K = 128
NCH = EPT // K
RPT = N // NS
GPT = N // NW
KG = 104


_ND = 13


def _deg_body(dst_hbm, zeros1_hbm, degp_hbm, *scr):
    dvv = list(scr[0:_ND])
    hist_v, rv0, rv1, acc_v = scr[_ND:4 + _ND]
    hist_sh = scr[4 + _ND]
    sems = list(scr[5 + _ND:])
    sdv = sems[0:_ND]
    semz, semz2, sr0, sr1 = sems[_ND:4 + _ND]
    semr = [sr0, sr1]
    rv = [rv0, rv1]
    c = lax.axis_index("c")
    s = lax.axis_index("s")
    wid = c * NS + s

    hz = pltpu.async_copy(zeros1_hbm.at[pl.ds(0, N)], hist_v, semz)
    hz2 = pltpu.async_copy(zeros1_hbm.at[pl.ds(0, RPT)], acc_v, semz2)

    hz.wait()
    ones = jnp.ones((16,), jnp.float32)

    def dgroup(g, carry):
        base = wid * EPT + g * (_ND * K)
        hd = []
        for u in range(_ND):
            hd.append(pltpu.async_copy(dst_hbm.at[pl.ds(base + u * K, K)],
                                       dvv[u], sdv[u]))
        for u in range(_ND):
            hd[u].wait()
            for j in range(K // 16):
                plsc.addupdate_scatter(hist_v,
                                       [dvv[u][pl.ds(j * 16, 16)]], ones)
        return carry
    lax.fori_loop(0, NCH // _ND, dgroup, 0)

    pltpu.sync_copy(hist_v, hist_sh.at[pl.ds(s * N, N)])
    plsc.subcore_barrier()
    hz2.wait()
    hr = [None] * (NS + 1)
    hr[0] = pltpu.async_copy(hist_sh.at[pl.ds(0 * N + s * RPT, RPT)],
                             rv[0], semr[0])
    for t in range(NS):
        hr[t].wait()
        if t + 1 < NS:
            hr[t + 1] = pltpu.async_copy(
                hist_sh.at[pl.ds((t + 1) * N + s * RPT, RPT)],
                rv[(t + 1) % 2], semr[(t + 1) % 2])
        buf = rv[t % 2]

        def addb(i, carry2):
            sl = pl.ds(i * 16, 16)
            acc_v[sl] = acc_v[sl] + buf[sl]
            return carry2
        lax.fori_loop(0, RPT // 16, addb, 0)
    pltpu.sync_copy(acc_v, degp_hbm.at[pl.ds(c * N + s * RPT, RPT)])


_NB = 3
_NG = 39
_NI = 6


def _propagate_body(hs_hbm, src_hbm, dst_hbm, zeros_hbm, accp_hbm, *scr):
    src_v = list(scr[0:_NI])
    dst_v = list(scr[_NI:2 * _NI])
    rows = list(scr[2 * _NI:2 * _NI + _NB])
    o = 2 * _NI + _NB
    sema = list(scr[o:o + _NI])
    semb = list(scr[o + _NI:o + 2 * _NI])
    semg = list(scr[o + 2 * _NI:o + 2 * _NI + _NB])
    semsc = list(scr[o + 2 * _NI + _NB:o + 2 * _NI + 2 * _NB])
    acc_sh = scr[-1]
    c = lax.axis_index("c")
    s = lax.axis_index("s")
    wid = c * NS + s
    pltpu.sync_copy(zeros_hbm.at[pl.ds(s * RPT, RPT)],
                    acc_sh.at[pl.ds(s * RPT, RPT)])
    plsc.subcore_barrier()

    def group(gi, carry):
        base = wid * EPT + gi * (_NG * K)
        ha = [None] * _NG
        hb = [None] * _NG
        hg = [None] * _NG
        hsc = [None] * _NG

        def sct(j):
            hg[j].wait()
            hb[j].wait()
            hsc[j] = pltpu.async_copy(rows[j % _NB],
                                      acc_sh.at[dst_v[j % _NI]],
                                      semsc[j % _NB], add=True)
        for j in range(_NB):
            off = base + j * K
            ha[j] = pltpu.async_copy(src_hbm.at[pl.ds(off, K)],
                                     src_v[j % _NI], sema[j % _NI])
            hb[j] = pltpu.async_copy(dst_hbm.at[pl.ds(off, K)],
                                     dst_v[j % _NI], semb[j % _NI])
        for j in range(_NG):
            rb = j % _NB
            if j >= _NB:
                hsc[j - _NB].wait()
            if j + _NB < _NG:
                off = base + (j + _NB) * K
                sl = (j + _NB) % _NI
                ha[j + _NB] = pltpu.async_copy(src_hbm.at[pl.ds(off, K)],
                                               src_v[sl], sema[sl])
                hb[j + _NB] = pltpu.async_copy(dst_hbm.at[pl.ds(off, K)],
                                               dst_v[sl], semb[sl])
            ha[j].wait()
            hg[j] = pltpu.async_copy(hs_hbm.at[src_v[j % _NI]],
                                     rows[rb], semg[rb])
            if j >= 1:
                sct(j - 1)
        sct(_NG - 1)
        for j in range(_NG - _NB, _NG):
            hsc[j].wait()
        return carry
    lax.fori_loop(0, NCH // _NG, group, 0)
    plsc.subcore_barrier()
    pltpu.sync_copy(acc_sh.at[pl.ds(s * RPT, RPT)],
                    accp_hbm.at[pl.ds(c * N + s * RPT, RPT)])


@functools.lru_cache(maxsize=None)
def _sc_kernels():
    mesh = plsc.VectorSubcoreMesh(core_axis_name="c", subcore_axis_name="s")
    deg = pl.kernel(
        _deg_body, mesh=mesh,
        out_type=jax.ShapeDtypeStruct((NC * N,), jnp.float32),
        compiler_params=pltpu.CompilerParams(needs_layout_passes=False),
        scratch_types=(
            [pltpu.VMEM((K,), jnp.int32)] * _ND
            + [pltpu.VMEM((N,), jnp.float32)]
            + [pltpu.VMEM((RPT,), jnp.float32)] * 3
            + [pltpu.VMEM_SHARED((NS * N,), jnp.float32)]
            + [pltpu.SemaphoreType.DMA] * (4 + _ND)))
    propagate = pl.kernel(
        _propagate_body, mesh=mesh,
        out_type=jax.ShapeDtypeStruct((NC * N, D), jnp.float32),
        scratch_types=(
            [pltpu.VMEM((K,), jnp.int32)] * (2 * _NI)
            + [pltpu.VMEM((K, D), jnp.float32)] * _NB
            + [pltpu.SemaphoreType.DMA] * (2 * _NI + 2 * _NB)
            + [pltpu.VMEM_SHARED((N, D), jnp.float32)]))
    return deg, propagate


def _sc_deg(dst, zeros1):
    return _sc_kernels()[0](dst, zeros1)


def _sc_propagate(hs, src, dst, zeros):
    return _sc_kernels()[1](hs, src, dst, zeros)


_R = 1248
_RF = 768


def _dinv_block(dega, degb):
    deg = dega[:, :1] + degb[:, :1] + 1.0
    return lax.rsqrt(deg)


def _tc_embed_body(idx_ref, embp_ref, w_ref, h0_ref, hw_ref):
    cids = lax.broadcasted_iota(jnp.int32, (_R, 16), 1)
    oh = (idx_ref[...] == cids).astype(jnp.float32)
    h0 = jnp.dot(oh, embp_ref[...], preferred_element_type=jnp.float32,
                 precision=lax.Precision.HIGHEST)
    h0_ref[...] = h0
    hw_ref[...] = jnp.dot(h0, w_ref[...], preferred_element_type=jnp.float32)


def _tc_scale_body(hw_ref, dega_ref, degb_ref, hs_ref):
    dinv = _dinv_block(dega_ref[...], degb_ref[...])
    hs_ref[...] = dinv * hw_ref[...]


def _tc_mid_body(acca_ref, accb_ref, hw_ref, hprev_ref, b_ref,
                 dega_ref, degb_ref, w_ref,
                 h_ref, hwn_ref, hsn_ref):
    dinv = _dinv_block(dega_ref[...], degb_ref[...])
    hw = hw_ref[...]
    h = dinv * (acca_ref[...] + accb_ref[...]) + dinv * dinv * hw \
        + b_ref[...] + hprev_ref[...]
    h = jnp.maximum(h, 0.0)
    h_ref[...] = h
    hwn = jnp.dot(h, w_ref[...], preferred_element_type=jnp.float32)
    hwn_ref[...] = hwn
    hsn_ref[...] = dinv * hwn


def _tc_final_body(acca_ref, accb_ref, hw_ref, hprev_ref, b_ref,
                   dega_ref, degb_ref, batch_ref,
                   fw1_ref, fb1_ref, fw2_ref, fb2_ref, ow_ref, ob_ref,
                   out_ref, pool_scr, cnt_scr):
    pid = pl.program_id(0)

    @pl.when(pid == 0)
    def _init():
        pool_scr[...] = jnp.zeros((G, D), jnp.float32)
        cnt_scr[...] = jnp.zeros((G, D), jnp.float32)

    dinv = _dinv_block(dega_ref[...], degb_ref[...])
    hw = hw_ref[...]
    h3 = dinv * (acca_ref[...] + accb_ref[...]) + dinv * dinv * hw \
        + b_ref[...] + hprev_ref[...]

    gids = lax.broadcasted_iota(jnp.int32, (_RF, G), 1)
    ohb = (batch_ref[...] == gids).astype(jnp.float32)
    pool_scr[...] += lax.dot_general(
        ohb, h3, (((0,), (0,)), ((), ())),
        preferred_element_type=jnp.float32,
        precision=lax.Precision.HIGHEST)
    cnt_scr[...] += jnp.sum(ohb, axis=0)[:, None]

    @pl.when(pid == pl.num_programs(0) - 1)
    def _mlp():
        pooled = pool_scr[...] / jnp.maximum(cnt_scr[...], 1.0)
        r1 = jnp.maximum(jnp.dot(pooled, fw1_ref[...],
                                 preferred_element_type=jnp.float32)
                         + fb1_ref[...], 0.0)
        r2 = jnp.maximum(jnp.dot(r1, fw2_ref[...],
                                 preferred_element_type=jnp.float32)
                         + fb2_ref[...], 0.0)
        out_ref[...] = jnp.dot(r2, ow_ref[...],
                               preferred_element_type=jnp.float32) + ob_ref[...]


def _row_spec(r, cols):
    return pl.BlockSpec((r, cols), lambda i: (i, 0))


def _rep_spec(shape):
    nd = len(shape)
    return pl.BlockSpec(shape, lambda i: (0,) * nd)


def _tc_embed(idxcol, embp, W):
    grid = N // _R
    return pl.pallas_call(
        _tc_embed_body,
        grid=(grid,),
        in_specs=[_row_spec(_R, 1), _rep_spec((16, D)), _rep_spec((D, D))],
        out_specs=[_row_spec(_R, D), _row_spec(_R, D)],
        out_shape=[jax.ShapeDtypeStruct((N, D), jnp.float32),
                   jax.ShapeDtypeStruct((N, D), jnp.float32)],
    )(idxcol, embp, W)


def _tc_scale(hw, dega, degb):
    grid = N // _R
    return pl.pallas_call(
        _tc_scale_body,
        grid=(grid,),
        in_specs=[_row_spec(_R, D), _row_spec(_R, 1), _row_spec(_R, 1)],
        out_specs=_row_spec(_R, D),
        out_shape=jax.ShapeDtypeStruct((N, D), jnp.float32),
    )(hw, dega, degb)


def _tc_mid(acca, accb, hw, hprev, b2d, dega, degb, Wn):
    grid = N // _R
    return pl.pallas_call(
        _tc_mid_body,
        grid=(grid,),
        in_specs=[_row_spec(_R, D), _row_spec(_R, D), _row_spec(_R, D),
                  _row_spec(_R, D), _rep_spec((1, D)),
                  _row_spec(_R, 1), _row_spec(_R, 1), _rep_spec((D, D))],
        out_specs=[_row_spec(_R, D), _row_spec(_R, D), _row_spec(_R, D)],
        out_shape=[jax.ShapeDtypeStruct((N, D), jnp.float32),
                   jax.ShapeDtypeStruct((N, D), jnp.float32),
                   jax.ShapeDtypeStruct((N, D), jnp.float32)],
    )(acca, accb, hw, hprev, b2d, dega, degb, Wn)


def _tc_final(acca, accb, hw, hprev, b2d, dega, degb, batch3,
              fcW1, fcb1, fcW2, fcb2, outWp, outb2):
    grid = N // _RF
    return pl.pallas_call(
        _tc_final_body,
        grid=(grid,),
        in_specs=[_row_spec(_RF, D), _row_spec(_RF, D), _row_spec(_RF, D),
                  _row_spec(_RF, D), _rep_spec((1, D)),
                  _row_spec(_RF, 1), _row_spec(_RF, 1),
                  _row_spec(_RF, 1),
                  _rep_spec((D, D)), _rep_spec((1, D)),
                  _rep_spec((D, G)), _rep_spec((1, G)),
                  _rep_spec((G, D)), _rep_spec((1, D))],
        out_specs=pl.BlockSpec((G, D), lambda i: (0, 0)),
        out_shape=jax.ShapeDtypeStruct((G, D), jnp.float32),
        scratch_shapes=[pltpu.VMEM((G, D), jnp.float32),
                        pltpu.VMEM((G, D), jnp.float32)],
    )(acca, accb, hw, hprev, b2d, dega, degb, batch3,
      fcW1, fcb1, fcW2, fcb2, outWp, outb2)


def kernel(x, edge_index, batch, emb, W1, b1, W2, b2, W3, b3,
           fcW1, fcb1, fcW2, fcb2, outW, outb):
    idx = jnp.nonzero(x, size=int(x.size), fill_value=0)[1].astype(jnp.int32)
    src = edge_index[0].astype(jnp.int32)
    dst = edge_index[1].astype(jnp.int32)

    zeros128 = jnp.zeros((N, D), jnp.float32)
    zeros1 = jnp.zeros((N,), jnp.float32)

    degp = _sc_deg(dst, zeros1)
    dega = degp[:N].reshape(N, 1)
    degb = degp[N:].reshape(N, 1)
    embp = jnp.pad(emb, ((0, 16 - emb.shape[0]), (0, 0)))
    h0, hw1 = _tc_embed(idx.reshape(N, 1), embp, W1)

    b1r = b1.reshape(1, D)
    b2r = b2.reshape(1, D)
    b3r = b3.reshape(1, D)
    batch3 = batch.astype(jnp.int32).reshape(N, 1)
    outWp = jnp.pad(outW, ((0, 0), (0, D - outW.shape[1])))
    outb2 = jnp.pad(outb.reshape(1, 1), ((0, 0), (0, D - 1)))

    hs1 = _tc_scale(hw1, dega, degb)

    accp1 = _sc_propagate(hs1, src, dst, zeros128)
    h1, hw2, hs2 = _tc_mid(accp1[:N], accp1[N:], hw1, h0, b1r, dega, degb, W2)

    accp2 = _sc_propagate(hs2, src, dst, zeros128)
    h2, hw3, hs3 = _tc_mid(accp2[:N], accp2[N:], hw2, h1, b2r, dega, degb, W3)

    accp3 = _sc_propagate(hs3, src, dst, zeros128)
    outp = _tc_final(accp3[:N], accp3[N:], hw3, h2, b3r, dega, degb, batch3,
                     fcW1, fcb1.reshape(1, D), fcW2,
                     jnp.pad(fcb2.reshape(1, G), ((0, 0), (0, 0))), outWp, outb2)
    return outp[:, :1]
